# Initial kernel scaffold; baseline (speedup 1.0000x reference)
import jax
import jax.numpy as jnp
from jax.experimental import pallas as pl


def kernel(x, edge_index, edge_attr, W1, b1, W2, b2, Wm1, bm1, Wm2, bm2):
    raise NotImplementedError("write your pallas kernel here")


# speedup vs baseline: 5.6881x; 5.6881x over previous
import functools

import jax
import jax.numpy as jnp
from jax import lax
from jax.experimental import pallas as pl
from jax.experimental.pallas import tpu as pltpu
from jax.experimental.pallas import tpu_sc as plsc

N = 100000
E = 1600000
HID = 64

NC = 2
NS = 16
NW = NC * NS

NP = 100352
RPT = NP // NS
ZR = 98

ER = 12800
EP = ER * 128
ROWS_W = ER // NW
ROWS_T = ER // NS


def _mesh():
    return plsc.VectorSubcoreMesh(core_axis_name="c", subcore_axis_name="s",
                                  num_cores=NC, num_subcores=NS)


def _fill(ref, rows, val):
    v = jnp.full((16,), val, ref.dtype)

    @pl.loop(0, rows)
    def _(i):
        ref[i, :] = v


def _zero_my_slice(acc, zbuf, s):
    @pl.loop(0, 64)
    def _(b):
        pltpu.sync_copy(zbuf, acc.at[pl.ds(s * RPT + b * ZR, ZR)])


def _sc_deg(dst2):
    @functools.partial(
        pl.kernel,
        out_type=jax.ShapeDtypeStruct((NC, NP, 16), jnp.float32),
        mesh=_mesh(),
        compiler_params=pltpu.CompilerParams(use_tc_tiling_on_sc=False, needs_layout_passes=False),
        scratch_types=[
            pltpu.VMEM_SHARED((NP, 16), jnp.float32),
            pltpu.VMEM((ZR, 16), jnp.float32),
            pltpu.VMEM((128, 16), jnp.float32),
            pltpu.VMEM((16, 128), jnp.int32),
            pltpu.SemaphoreType.DMA,
        ],
    )
    def k(dst_hbm, out_hbm, acc, zbuf, ones, dstv, sem):
        c = lax.axis_index("c")
        s = lax.axis_index("s")
        _fill(zbuf, ZR, 0.0)
        _fill(ones, 128, 1.0)
        _zero_my_slice(acc, zbuf, s)
        plsc.subcore_barrier()

        @pl.loop(0, ER // NC // NS // 16)
        def _(blk):
            row0 = c * (ER // NC) + s * (ER // NC // NS) + blk * 16
            pltpu.sync_copy(dst_hbm.at[pl.ds(row0, 16)], dstv)
            descs = [pltpu.async_copy(ones, acc.at[dstv.at[j]], sem, add=True)
                     for j in range(16)]
            for d in descs:
                d.wait()

        plsc.subcore_barrier()
        pltpu.sync_copy(acc.at[pl.ds(s * RPT, RPT)],
                        out_hbm.at[c, pl.ds(s * RPT, RPT)])

    return k(dst2)


def _sc_segsum(src2, dst2, gflat):
    @functools.partial(
        pl.kernel,
        out_type=jax.ShapeDtypeStruct((4, NP, 16), jnp.float32),
        mesh=_mesh(),
        compiler_params=pltpu.CompilerParams(use_tc_tiling_on_sc=False, needs_layout_passes=False),
        scratch_types=[
            pltpu.VMEM_SHARED((NP, 16), jnp.float32),
            pltpu.VMEM((ZR, 16), jnp.float32),
            pltpu.VMEM((4, 128), jnp.int32),
            pltpu.VMEM((4, 128), jnp.int32),
            pltpu.VMEM((4, 128), jnp.int32),
            pltpu.VMEM((4, 128, 16), jnp.float32),
            pltpu.SemaphoreType.DMA,
            pltpu.SemaphoreType.DMA,
        ],
    )
    def k(src_hbm, dst_hbm, g_hbm, out_hbm,
          acc, zbuf, srcv, dstv, gidx, grow, semg, sems):
        c = lax.axis_index("c")
        s = lax.axis_index("s")
        _fill(zbuf, ZR, 0.0)

        for cgl in range(2):
            cg = c * 2 + cgl
            _zero_my_slice(acc, zbuf, s)
            plsc.subcore_barrier()

            @pl.loop(0, ROWS_T // 4)
            def _(blk):
                row0 = s * ROWS_T + blk * 4
                pltpu.sync_copy(src_hbm.at[pl.ds(row0, 4)], srcv)
                pltpu.sync_copy(dst_hbm.at[pl.ds(row0, 4)], dstv)

                @pl.loop(0, 4)
                def _(i):
                    for m in range(8):
                        sl = pl.ds(m * 16, 16)
                        gidx[i, sl] = srcv[i, sl] * 4 + cg

                dg = [pltpu.async_copy(g_hbm.at[gidx.at[j]], grow.at[j], semg)
                      for j in range(4)]
                for d in dg:
                    d.wait()
                ds_ = [pltpu.async_copy(grow.at[j], acc.at[dstv.at[j]], sems,
                                        add=True)
                       for j in range(4)]
                for d in ds_:
                    d.wait()

            plsc.subcore_barrier()
            pltpu.sync_copy(acc.at[pl.ds(s * RPT, RPT)],
                            out_hbm.at[cg, pl.ds(s * RPT, RPT)])

    return k(src2, dst2, gflat)


def _sc_mlp(src2, dst2, p, q, r, wvec):
    BR = 4
    BE = BR * 128

    @functools.partial(
        pl.kernel,
        out_type=jax.ShapeDtypeStruct((EP,), jnp.float32),
        mesh=_mesh(),
        compiler_params=pltpu.CompilerParams(use_tc_tiling_on_sc=False, needs_layout_passes=False),
        scratch_types=[
            pltpu.VMEM((BR, 128), jnp.int32),
            pltpu.VMEM((BR, 128), jnp.int32),
            pltpu.VMEM((BE, HID), jnp.float32),
            pltpu.VMEM((BE, HID), jnp.float32),
            pltpu.VMEM((BE, HID), jnp.float32),
            pltpu.VMEM((BE,), jnp.float32),
            pltpu.VMEM((72,), jnp.float32),
            pltpu.SemaphoreType.DMA,
            pltpu.SemaphoreType.DMA,
        ],
    )
    def k(src_hbm, dst_hbm, p_hbm, q_hbm, r_hbm, w_hbm, out_hbm,
          srcv, dstv, ps, qd, rv, outv, wv, semg, semr):
        c = lax.axis_index("c")
        s = lax.axis_index("s")
        wid = s * NC + c
        pltpu.sync_copy(w_hbm, wv)
        wk = [wv[pl.ds(16 * t, 16)] for t in range(4)]
        bm2s = wv[pl.ds(56, 16)][8]
        lane = lax.iota(jnp.int32, 16)

        @pl.loop(0, ROWS_W // BR)
        def _(blk):
            row0 = wid * ROWS_W + blk * BR
            e0 = row0 * 128
            pltpu.sync_copy(src_hbm.at[pl.ds(row0, BR)], srcv)
            pltpu.sync_copy(dst_hbm.at[pl.ds(row0, BR)], dstv)
            dr = pltpu.async_copy(r_hbm.at[pl.ds(e0, BE)], rv, semr)
            dp = [pltpu.async_copy(p_hbm.at[srcv.at[j]],
                                   ps.at[pl.ds(j * 128, 128)], semg)
                  for j in range(BR)]
            dq = [pltpu.async_copy(q_hbm.at[dstv.at[j]],
                                   qd.at[pl.ds(j * 128, 128)], semg)
                  for j in range(BR)]
            for d in dp + dq:
                d.wait()
            dr.wait()

            @pl.loop(0, BE // 16)
            def _(grp):
                res = jnp.full((16,), 0.0, jnp.float32) + bm2s
                for l in range(16):
                    e = grp * 16 + l
                    acc = None
                    for t in range(4):
                        sl = pl.ds(16 * t, 16)
                        z = ps[e, sl] + qd[e, sl] + rv[e, sl]
                        z = jnp.maximum(z, 0.0)
                        zt = z * wk[t]
                        acc = zt if acc is None else acc + zt
                    res = jnp.where(lane == l, res + jnp.sum(acc), res)
                outv[pl.ds(grp * 16, 16)] = res

            pltpu.sync_copy(outv, out_hbm.at[pl.ds(e0, BE)])

    return k(src2, dst2, p, q, r, wvec)


_NBLK = 1024
_NGRID = NP // _NBLK


def _tc_prep(x_pad, deg_parts, W1):
    def body(x_ref, dp_ref, w_ref, g_ref, dinv_ref):
        deg = dp_ref[0, :, 0:1] + dp_ref[1, :, 0:1] + 1.0
        dinv = lax.rsqrt(deg)
        h = jnp.dot(x_ref[...], w_ref[...], preferred_element_type=jnp.float32)
        g_ref[...] = h * dinv
        dinv_ref[...] = dinv

    return pl.pallas_call(
        body,
        grid=(_NGRID,),
        in_specs=[
            pl.BlockSpec((_NBLK, 4), lambda i: (i, 0)),
            pl.BlockSpec((NC, _NBLK, 16), lambda i: (0, i, 0)),
            pl.BlockSpec((4, HID), lambda i: (0, 0)),
        ],
        out_specs=[
            pl.BlockSpec((_NBLK, HID), lambda i: (i, 0)),
            pl.BlockSpec((_NBLK, 1), lambda i: (i, 0)),
        ],
        out_shape=[
            jax.ShapeDtypeStruct((NP, HID), jnp.float32),
            jax.ShapeDtypeStruct((NP, 1), jnp.float32),
        ],
    )(x_pad, deg_parts, W1)


def _tc_combine(S, g, dinv, b_row, Wn):
    def body(s_ref, g_ref, dinv_ref, b_ref, w_ref, out_ref):
        Sb = jnp.concatenate([s_ref[i] for i in range(4)], axis=1)
        dinv = dinv_ref[...]
        h = jnp.maximum(dinv * (Sb + g_ref[...]) + b_ref[...], 0.0)
        out_ref[...] = dinv * jnp.dot(h, w_ref[...],
                                      preferred_element_type=jnp.float32)

    return pl.pallas_call(
        body,
        grid=(_NGRID,),
        in_specs=[
            pl.BlockSpec((4, _NBLK, 16), lambda i: (0, i, 0)),
            pl.BlockSpec((_NBLK, HID), lambda i: (i, 0)),
            pl.BlockSpec((_NBLK, 1), lambda i: (i, 0)),
            pl.BlockSpec((1, HID), lambda i: (0, 0)),
            pl.BlockSpec((HID, HID), lambda i: (0, 0)),
        ],
        out_specs=pl.BlockSpec((_NBLK, HID), lambda i: (i, 0)),
        out_shape=jax.ShapeDtypeStruct((NP, HID), jnp.float32),
    )(S, g, dinv, b_row, Wn)


def _tc_final_nodes(S, g, dinv, b_row, Wsrc, Wdst):
    def body(s_ref, g_ref, dinv_ref, b_ref, ws_ref, wd_ref, p_ref, q_ref):
        Sb = jnp.concatenate([s_ref[i] for i in range(4)], axis=1)
        dinv = dinv_ref[...]
        h = jnp.maximum(dinv * (Sb + g_ref[...]) + b_ref[...], 0.0)
        p_ref[...] = jnp.dot(h, ws_ref[...], preferred_element_type=jnp.float32)
        q_ref[...] = jnp.dot(h, wd_ref[...], preferred_element_type=jnp.float32)

    return pl.pallas_call(
        body,
        grid=(_NGRID,),
        in_specs=[
            pl.BlockSpec((4, _NBLK, 16), lambda i: (0, i, 0)),
            pl.BlockSpec((_NBLK, HID), lambda i: (i, 0)),
            pl.BlockSpec((_NBLK, 1), lambda i: (i, 0)),
            pl.BlockSpec((1, HID), lambda i: (0, 0)),
            pl.BlockSpec((HID, HID), lambda i: (0, 0)),
            pl.BlockSpec((HID, HID), lambda i: (0, 0)),
        ],
        out_specs=[
            pl.BlockSpec((_NBLK, HID), lambda i: (i, 0)),
            pl.BlockSpec((_NBLK, HID), lambda i: (i, 0)),
        ],
        out_shape=[
            jax.ShapeDtypeStruct((NP, HID), jnp.float32),
            jax.ShapeDtypeStruct((NP, HID), jnp.float32),
        ],
    )(S, g, dinv, b_row, Wsrc, Wdst)


_EBLK = 2048


def _tc_edge_r(ea_pad, We, bm1_row):
    def body(ea_ref, w_ref, b_ref, r_ref):
        r_ref[...] = jnp.dot(ea_ref[...], w_ref[...],
                             preferred_element_type=jnp.float32) + b_ref[...]

    return pl.pallas_call(
        body,
        grid=(EP // _EBLK,),
        in_specs=[
            pl.BlockSpec((_EBLK, 4), lambda i: (i, 0)),
            pl.BlockSpec((4, HID), lambda i: (0, 0)),
            pl.BlockSpec((1, HID), lambda i: (0, 0)),
        ],
        out_specs=pl.BlockSpec((_EBLK, HID), lambda i: (i, 0)),
        out_shape=jax.ShapeDtypeStruct((EP, HID), jnp.float32),
    )(ea_pad, We, bm1_row)


def kernel(x, edge_index, edge_attr, W1, b1, W2, b2, Wm1, bm1, Wm2, bm2):
    src = edge_index[0]
    dst = edge_index[1]
    src2 = jnp.full((EP,), N, jnp.int32).at[:E].set(src).reshape(ER, 128)
    dst2 = jnp.full((EP,), N, jnp.int32).at[:E].set(dst).reshape(ER, 128)
    ea_pad = jnp.zeros((EP, 4), jnp.float32).at[:E].set(edge_attr)
    x_pad = jnp.zeros((NP, 4), jnp.float32).at[:N].set(x)
    b1_row = b1.reshape(1, HID)
    b2_row = b2.reshape(1, HID)
    bm1_row = bm1.reshape(1, HID)
    Wsrc = Wm1[0:HID]
    Wdst = Wm1[HID:2 * HID]
    We = Wm1[2 * HID:]
    wvec = jnp.zeros((72,), jnp.float32).at[0:HID].set(Wm2[:, 0]).at[HID].set(bm2[0])

    r = _tc_edge_r(ea_pad, We, bm1_row)
    deg_parts = _sc_deg(dst2)
    g1, dinv = _tc_prep(x_pad, deg_parts, W1)
    S1 = _sc_segsum(src2, dst2, g1.reshape(NP * 4, 16))
    g2 = _tc_combine(S1, g1, dinv, b1_row, W2)
    S2 = _sc_segsum(src2, dst2, g2.reshape(NP * 4, 16))
    p, q = _tc_final_nodes(S2, g2, dinv, b2_row, Wsrc, Wdst)
    logits = _sc_mlp(src2, dst2, p, q, r, wvec)
    return logits[:E]


# speedup vs baseline: 6.9063x; 1.2142x over previous
import functools

import jax
import jax.numpy as jnp
from jax import lax
from jax.experimental import pallas as pl
from jax.experimental.pallas import tpu as pltpu
from jax.experimental.pallas import tpu_sc as plsc

N = 100000
E = 1600000
HID = 64

NC = 2
NS = 16
NW = NC * NS

NP = 100352
RPT = NP // NS
ZR = 98

ER = 12800
EP = ER * 128
ROWS_W = ER // NW
ROWS_T = ER // NS

_SC_PARAMS = dict(
    compiler_params=pltpu.CompilerParams(use_tc_tiling_on_sc=False,
                                         needs_layout_passes=False),
)


def _mesh():
    return plsc.VectorSubcoreMesh(core_axis_name="c", subcore_axis_name="s",
                                  num_cores=NC, num_subcores=NS)


def _fill(ref, rows, val):
    v = jnp.full((16,), val, ref.dtype)

    @pl.loop(0, rows)
    def _(i):
        ref[i, :] = v


def _zero_my_slice(acc, zbuf, s):
    @pl.loop(0, 64)
    def _(b):
        pltpu.sync_copy(zbuf, acc.at[pl.ds(s * RPT + b * ZR, ZR)])


def _sc_deg(dst2):
    @functools.partial(
        pl.kernel,
        out_type=jax.ShapeDtypeStruct((NC, NP, 16), jnp.float32),
        mesh=_mesh(),
        scratch_types=[
            pltpu.VMEM_SHARED((NP, 16), jnp.float32),
            pltpu.VMEM((ZR, 16), jnp.float32),
            pltpu.VMEM((128, 16), jnp.float32),
            pltpu.VMEM((16, 128), jnp.int32),
            pltpu.SemaphoreType.DMA,
        ],
        **_SC_PARAMS,
    )
    def k(dst_hbm, out_hbm, acc, zbuf, ones, dstv, sem):
        c = lax.axis_index("c")
        s = lax.axis_index("s")
        _fill(zbuf, ZR, 0.0)
        _fill(ones, 128, 1.0)
        _zero_my_slice(acc, zbuf, s)
        plsc.subcore_barrier()

        @pl.loop(0, ER // NC // NS // 16)
        def _(blk):
            row0 = c * (ER // NC) + s * (ER // NC // NS) + blk * 16
            pltpu.sync_copy(dst_hbm.at[pl.ds(row0, 16)], dstv)
            descs = [pltpu.async_copy(ones, acc.at[dstv.at[j]], sem, add=True)
                     for j in range(16)]
            for d in descs:
                d.wait()

        plsc.subcore_barrier()
        pltpu.sync_copy(acc.at[pl.ds(s * RPT, RPT)],
                        out_hbm.at[c, pl.ds(s * RPT, RPT)])

    return k(dst2)


def _sc_segsum(src2, dst2, gflat):
    BR = 4
    NBLK = ROWS_T // BR
    HALF = NBLK // 2

    @functools.partial(
        pl.kernel,
        out_type=jax.ShapeDtypeStruct((4, NP, 16), jnp.float32),
        mesh=_mesh(),
        scratch_types=[
            pltpu.VMEM_SHARED((NP, 16), jnp.float32),
            pltpu.VMEM((ZR, 16), jnp.float32),
            pltpu.VMEM((BR, 128), jnp.int32), pltpu.VMEM((BR, 128), jnp.int32),
            pltpu.VMEM((BR, 128), jnp.int32), pltpu.VMEM((BR, 128), jnp.int32),
            pltpu.VMEM((BR, 128), jnp.int32), pltpu.VMEM((BR, 128), jnp.int32),
            pltpu.VMEM((BR, 128), jnp.int32), pltpu.VMEM((BR, 128), jnp.int32),
            pltpu.VMEM((BR, 128, 16), jnp.float32),
            pltpu.VMEM((BR, 128, 16), jnp.float32),
            pltpu.SemaphoreType.DMA, pltpu.SemaphoreType.DMA,
            pltpu.SemaphoreType.DMA, pltpu.SemaphoreType.DMA,
            pltpu.SemaphoreType.DMA,
        ],
        **_SC_PARAMS,
    )
    def k(src_hbm, dst_hbm, g_hbm, out_hbm, acc, zbuf,
          srcv0, srcv1, dstv0, dstv1, gidx0, gidx1, dstx0, dstx1,
          grow0, grow1, semi0, semi1, semg0, semg1, sems):
        c = lax.axis_index("c")
        s = lax.axis_index("s")
        srcv = (srcv0, srcv1)
        dstv = (dstv0, dstv1)
        gidx = (gidx0, gidx1)
        dstx = (dstx0, dstx1)
        grow = (grow0, grow1)
        semi = (semi0, semi1)
        semg = (semg0, semg1)
        _fill(zbuf, ZR, 0.0)

        def rowbase(b):
            return s * ROWS_T + b * BR

        def fire_idx(b, sl):
            r0 = rowbase(b)
            pltpu.async_copy(src_hbm.at[pl.ds(r0, BR)], srcv[sl], semi[sl])
            pltpu.async_copy(dst_hbm.at[pl.ds(r0, BR)], dstv[sl], semi[sl])

        def wait_g(sl):
            for j in range(BR):
                pltpu.make_async_copy(g_hbm.at[gidx[sl].at[j]],
                                      grow[sl].at[j], semg[sl]).wait()

        def scatter(sl):
            ds_ = [pltpu.async_copy(grow[sl].at[j], acc.at[dstx[sl].at[j]],
                                    sems, add=True)
                   for j in range(BR)]
            for d in ds_:
                d.wait()

        for cgl in range(2):
            cg = c * 2 + cgl

            def fire(b, sl, cg=cg):
                pltpu.make_async_copy(src_hbm.at[pl.ds(0, BR)], srcv[sl],
                                      semi[sl]).wait()
                pltpu.make_async_copy(dst_hbm.at[pl.ds(0, BR)], dstv[sl],
                                      semi[sl]).wait()
                for i in range(BR):
                    for m in range(8):
                        slc = pl.ds(m * 16, 16)
                        gidx[sl][i, slc] = srcv[sl][i, slc] * 4 + cg
                        dstx[sl][i, slc] = dstv[sl][i, slc]
                for j in range(BR):
                    pltpu.async_copy(g_hbm.at[gidx[sl].at[j]], grow[sl].at[j],
                                     semg[sl])

            _zero_my_slice(acc, zbuf, s)
            plsc.subcore_barrier()
            fire_idx(0, 0)
            fire(0, 0)
            fire_idx(1, 1)

            @pl.loop(0, HALF)
            def _(kk):
                b0 = kk * 2
                fire(b0 + 1, 1)
                not_last = kk < HALF - 1

                @pl.when(not_last)
                def _():
                    fire_idx(b0 + 2, 0)

                wait_g(0)
                scatter(0)

                @pl.when(not_last)
                def _():
                    fire(b0 + 2, 0)
                    fire_idx(b0 + 3, 1)

                wait_g(1)
                scatter(1)

            plsc.subcore_barrier()
            pltpu.sync_copy(acc.at[pl.ds(s * RPT, RPT)],
                            out_hbm.at[cg, pl.ds(s * RPT, RPT)])

    return k(src2, dst2, gflat)


def _sc_mlp(src2, dst2, pb, qb, rb, wvec):
    BR = 4
    BE = BR * 128
    NBLK = ROWS_W // BR
    HALF = NBLK // 2

    @functools.partial(
        pl.kernel,
        out_type=jax.ShapeDtypeStruct((EP,), jnp.float32),
        mesh=_mesh(),
        scratch_types=[
            pltpu.VMEM((BR, 128), jnp.int32), pltpu.VMEM((BR, 128), jnp.int32),
            pltpu.VMEM((BR, 128), jnp.int32), pltpu.VMEM((BR, 128), jnp.int32),
            pltpu.VMEM((BR, 128), jnp.int32), pltpu.VMEM((BR, 128), jnp.int32),
            pltpu.VMEM((BR, 128), jnp.int32), pltpu.VMEM((BR, 128), jnp.int32),
            pltpu.VMEM((BE, HID), jnp.bfloat16),
            pltpu.VMEM((BE, HID), jnp.bfloat16),
            pltpu.VMEM((BE, HID), jnp.bfloat16),
            pltpu.VMEM((BE, HID), jnp.bfloat16),
            pltpu.VMEM((BE // 2, 128), jnp.bfloat16),
            pltpu.VMEM((BE // 2, 128), jnp.bfloat16),
            pltpu.VMEM((BE,), jnp.float32), pltpu.VMEM((BE,), jnp.float32),
            pltpu.VMEM((72,), jnp.float32),
            pltpu.SemaphoreType.DMA, pltpu.SemaphoreType.DMA,
            pltpu.SemaphoreType.DMA, pltpu.SemaphoreType.DMA,
        ],
        **_SC_PARAMS,
    )
    def k(src_hbm, dst_hbm, p_hbm, q_hbm, r_hbm, w_hbm, out_hbm,
          srcv0, srcv1, dstv0, dstv1, sidx0, sidx1, didx0, didx1,
          ps0, ps1, qd0, qd1, rv0, rv1,
          outv0, outv1, wv, semi0, semi1, semg0, semg1):
        c = lax.axis_index("c")
        s = lax.axis_index("s")
        wid = s * NC + c
        srcv = (srcv0, srcv1)
        dstv = (dstv0, dstv1)
        sidx = (sidx0, sidx1)
        didx = (didx0, didx1)
        psb = (ps0, ps1)
        qdb = (qd0, qd1)
        rv = (rv0, rv1)
        outv = (outv0, outv1)
        semi = (semi0, semi1)
        semg = (semg0, semg1)

        pltpu.sync_copy(w_hbm, wv)
        w_vc = [wv[pl.ds(16 * t, 16)] for t in range(4)]
        bm2s = wv[pl.ds(56, 16)][8]
        lane = lax.iota(jnp.int32, 16)

        def rowbase(b):
            return wid * ROWS_W + b * BR

        def fire_idx(b, sl):
            r0 = rowbase(b)
            pltpu.async_copy(src_hbm.at[pl.ds(r0, BR)], srcv[sl], semi[sl])
            pltpu.async_copy(dst_hbm.at[pl.ds(r0, BR)], dstv[sl], semi[sl])

        def fire(b, sl):
            pltpu.make_async_copy(src_hbm.at[pl.ds(0, BR)], srcv[sl],
                                  semi[sl]).wait()
            pltpu.make_async_copy(dst_hbm.at[pl.ds(0, BR)], dstv[sl],
                                  semi[sl]).wait()
            r0 = rowbase(b)
            for i in range(BR):
                for m in range(8):
                    slc = pl.ds(m * 16, 16)
                    sidx[sl][i, slc] = srcv[sl][i, slc]
                    didx[sl][i, slc] = dstv[sl][i, slc]
            for j in range(BR):
                pltpu.async_copy(p_hbm.at[sidx[sl].at[j]],
                                 psb[sl].at[pl.ds(j * 128, 128)], semg[sl])
            for j in range(BR):
                pltpu.async_copy(q_hbm.at[didx[sl].at[j]],
                                 qdb[sl].at[pl.ds(j * 128, 128)], semg[sl])
            pltpu.async_copy(r_hbm.at[pl.ds(r0 * 64, BE // 2)], rv[sl],
                             semg[sl])

        def wait_all(sl):
            pltpu.make_async_copy(p_hbm.at[pl.ds(0, BE)], psb[sl],
                                  semg[sl]).wait()
            pltpu.make_async_copy(q_hbm.at[pl.ds(0, BE)], qdb[sl],
                                  semg[sl]).wait()
            pltpu.make_async_copy(r_hbm.at[pl.ds(0, BE // 2)], rv[sl],
                                  semg[sl]).wait()

        def compute(b, sl):
            @pl.loop(0, BE // 16)
            def _(grp):
                res = jnp.zeros((16,), jnp.float32) + bm2s
                for l in range(16):
                    e = grp * 16 + l
                    row = grp * 8 + (l // 2)
                    off = (l % 2) * 64
                    acc = None
                    for t in range(2):
                        sp = pl.ds(32 * t, 32)
                        sv = (psb[sl][e, sp] + qdb[sl][e, sp]
                              + rv[sl][row, pl.ds(off + 32 * t, 32)])
                        sv = jnp.maximum(sv, jnp.bfloat16(0.0))
                        ae, ao = plsc.unpack(
                            sv, format=plsc.PackFormat.INTERLEAVED)
                        pa = ae * w_vc[2 * t] + ao * w_vc[2 * t + 1]
                        acc = pa if acc is None else acc + pa
                    res = jnp.where(lane == l, res + jnp.sum(acc), res)
                outv[sl][pl.ds(grp * 16, 16)] = res

            pltpu.sync_copy(outv[sl], out_hbm.at[pl.ds(rowbase(b) * 128, BE)])

        fire_idx(0, 0)
        fire(0, 0)
        fire_idx(1, 1)

        @pl.loop(0, HALF)
        def _(kk):
            b0 = kk * 2
            fire(b0 + 1, 1)
            not_last = kk < HALF - 1

            @pl.when(not_last)
            def _():
                fire_idx(b0 + 2, 0)

            wait_all(0)
            compute(b0, 0)

            @pl.when(not_last)
            def _():
                fire(b0 + 2, 0)
                fire_idx(b0 + 3, 1)

            wait_all(1)
            compute(b0 + 1, 1)

    return k(src2, dst2, pb, qb, rb, wvec)


_NBLK = 1024
_NGRID = NP // _NBLK


def _tc_prep(x_pad, deg_parts, W1):
    def body(x_ref, dp_ref, w_ref, g_ref, dinv_ref):
        deg = dp_ref[0, :, 0:1] + dp_ref[1, :, 0:1] + 1.0
        dinv = lax.rsqrt(deg)
        h = jnp.dot(x_ref[...], w_ref[...], preferred_element_type=jnp.float32)
        g_ref[...] = h * dinv
        dinv_ref[...] = dinv

    return pl.pallas_call(
        body,
        grid=(_NGRID,),
        in_specs=[
            pl.BlockSpec((_NBLK, 4), lambda i: (i, 0)),
            pl.BlockSpec((NC, _NBLK, 16), lambda i: (0, i, 0)),
            pl.BlockSpec((4, HID), lambda i: (0, 0)),
        ],
        out_specs=[
            pl.BlockSpec((_NBLK, HID), lambda i: (i, 0)),
            pl.BlockSpec((_NBLK, 1), lambda i: (i, 0)),
        ],
        out_shape=[
            jax.ShapeDtypeStruct((NP, HID), jnp.float32),
            jax.ShapeDtypeStruct((NP, 1), jnp.float32),
        ],
    )(x_pad, deg_parts, W1)


def _tc_combine(S, g, dinv, b_row, Wn):
    def body(s_ref, g_ref, dinv_ref, b_ref, w_ref, out_ref):
        Sb = jnp.concatenate([s_ref[i] for i in range(4)], axis=1)
        dinv = dinv_ref[...]
        h = jnp.maximum(dinv * (Sb + g_ref[...]) + b_ref[...], 0.0)
        out_ref[...] = dinv * jnp.dot(h, w_ref[...],
                                      preferred_element_type=jnp.float32)

    return pl.pallas_call(
        body,
        grid=(_NGRID,),
        in_specs=[
            pl.BlockSpec((4, _NBLK, 16), lambda i: (0, i, 0)),
            pl.BlockSpec((_NBLK, HID), lambda i: (i, 0)),
            pl.BlockSpec((_NBLK, 1), lambda i: (i, 0)),
            pl.BlockSpec((1, HID), lambda i: (0, 0)),
            pl.BlockSpec((HID, HID), lambda i: (0, 0)),
        ],
        out_specs=pl.BlockSpec((_NBLK, HID), lambda i: (i, 0)),
        out_shape=jax.ShapeDtypeStruct((NP, HID), jnp.float32),
    )(S, g, dinv, b_row, Wn)


def _tc_final_nodes(S, g, dinv, b_row, Wsrc, Wdst):
    def body(s_ref, g_ref, dinv_ref, b_ref, ws_ref, wd_ref, p_ref, q_ref):
        Sb = jnp.concatenate([s_ref[i] for i in range(4)], axis=1)
        dinv = dinv_ref[...]
        h = jnp.maximum(dinv * (Sb + g_ref[...]) + b_ref[...], 0.0)
        p_ref[...] = jnp.dot(h, ws_ref[...],
                             preferred_element_type=jnp.float32
                             ).astype(jnp.bfloat16)
        q_ref[...] = jnp.dot(h, wd_ref[...],
                             preferred_element_type=jnp.float32
                             ).astype(jnp.bfloat16)

    return pl.pallas_call(
        body,
        grid=(_NGRID,),
        in_specs=[
            pl.BlockSpec((4, _NBLK, 16), lambda i: (0, i, 0)),
            pl.BlockSpec((_NBLK, HID), lambda i: (i, 0)),
            pl.BlockSpec((_NBLK, 1), lambda i: (i, 0)),
            pl.BlockSpec((1, HID), lambda i: (0, 0)),
            pl.BlockSpec((HID, HID), lambda i: (0, 0)),
            pl.BlockSpec((HID, HID), lambda i: (0, 0)),
        ],
        out_specs=[
            pl.BlockSpec((_NBLK, HID), lambda i: (i, 0)),
            pl.BlockSpec((_NBLK, HID), lambda i: (i, 0)),
        ],
        out_shape=[
            jax.ShapeDtypeStruct((NP, HID), jnp.bfloat16),
            jax.ShapeDtypeStruct((NP, HID), jnp.bfloat16),
        ],
    )(S, g, dinv, b_row, Wsrc, Wdst)


_EBLK = 1024


def _tc_edge_r(ea8, We, bm1_row):
    def body(ea_ref, w_ref, b_ref, r_ref):
        w = w_ref[...]
        b = b_ref[...]
        r0 = jnp.dot(ea_ref[:, 0:4], w, preferred_element_type=jnp.float32) + b
        r1 = jnp.dot(ea_ref[:, 4:8], w, preferred_element_type=jnp.float32) + b
        r_ref[...] = jnp.concatenate([r0, r1], axis=1).astype(jnp.bfloat16)

    return pl.pallas_call(
        body,
        grid=(EP // 2 // _EBLK,),
        in_specs=[
            pl.BlockSpec((_EBLK, 8), lambda i: (i, 0)),
            pl.BlockSpec((4, HID), lambda i: (0, 0)),
            pl.BlockSpec((1, HID), lambda i: (0, 0)),
        ],
        out_specs=pl.BlockSpec((_EBLK, 128), lambda i: (i, 0)),
        out_shape=jax.ShapeDtypeStruct((EP // 2, 128), jnp.bfloat16),
    )(ea8, We, bm1_row)


def kernel(x, edge_index, edge_attr, W1, b1, W2, b2, Wm1, bm1, Wm2, bm2):
    src = edge_index[0]
    dst = edge_index[1]
    src2 = jnp.full((EP,), N, jnp.int32).at[:E].set(src).reshape(ER, 128)
    dst2 = jnp.full((EP,), N, jnp.int32).at[:E].set(dst).reshape(ER, 128)
    ea8 = jnp.zeros((EP, 4), jnp.float32).at[:E].set(edge_attr).reshape(EP // 2, 8)
    x_pad = jnp.zeros((NP, 4), jnp.float32).at[:N].set(x)
    b1_row = b1.reshape(1, HID)
    b2_row = b2.reshape(1, HID)
    bm1_row = bm1.reshape(1, HID)
    Wsrc = Wm1[0:HID]
    Wdst = Wm1[HID:2 * HID]
    We = Wm1[2 * HID:]
    perm = (list(range(0, 32, 2)) + list(range(1, 32, 2))
            + list(range(32, 64, 2)) + list(range(33, 64, 2)))
    wvec = (jnp.zeros((72,), jnp.float32)
            .at[0:HID].set(Wm2[jnp.array(perm), 0])
            .at[HID].set(bm2[0]))

    rb = _tc_edge_r(ea8, We, bm1_row)
    deg_parts = _sc_deg(dst2)
    g1, dinv = _tc_prep(x_pad, deg_parts, W1)
    S1 = _sc_segsum(src2, dst2, g1.reshape(NP * 4, 16))
    g2 = _tc_combine(S1, g1, dinv, b1_row, W2)
    S2 = _sc_segsum(src2, dst2, g2.reshape(NP * 4, 16))
    p, q = _tc_final_nodes(S2, g2, dinv, b2_row, Wsrc, Wdst)
    logits = _sc_mlp(src2, dst2, p, q, rb, wvec)
    return logits[:E]


# speedup vs baseline: 7.4260x; 1.0753x over previous
import functools

import jax
import jax.numpy as jnp
from jax import lax
from jax.experimental import pallas as pl
from jax.experimental.pallas import tpu as pltpu
from jax.experimental.pallas import tpu_sc as plsc

N = 100000
E = 1600000
HID = 64

NC = 2
NS = 16
NW = NC * NS

NP = 100352
RPT = NP // NS
ZR = 98
NPW = NP // NW
NCH = 448

ER = 12800
EP = ER * 128
ROWS_W = ER // NW
ROWS_T = ER // NS

_SC_PARAMS = dict(
    compiler_params=pltpu.CompilerParams(use_tc_tiling_on_sc=False,
                                         needs_layout_passes=False),
)


def _mesh():
    return plsc.VectorSubcoreMesh(core_axis_name="c", subcore_axis_name="s",
                                  num_cores=NC, num_subcores=NS)


def _fill(ref, rows, val):
    v = jnp.full((16,), val, ref.dtype)

    @pl.loop(0, rows)
    def _(i):
        ref[i, :] = v


def _zero_my_slice(acc, zbuf, s):
    @pl.loop(0, 64)
    def _(b):
        pltpu.sync_copy(zbuf, acc.at[pl.ds(s * RPT + b * ZR, ZR)])


def _sc_deg(dst2):
    @functools.partial(
        pl.kernel,
        out_type=jax.ShapeDtypeStruct((NC, NP, 16), jnp.float32),
        mesh=_mesh(),
        scratch_types=[
            pltpu.VMEM_SHARED((NP, 16), jnp.float32),
            pltpu.VMEM((ZR, 16), jnp.float32),
            pltpu.VMEM((128, 16), jnp.float32),
            pltpu.VMEM((16, 128), jnp.int32),
            pltpu.SemaphoreType.DMA,
        ],
        **_SC_PARAMS,
    )
    def k(dst_hbm, out_hbm, acc, zbuf, ones, dstv, sem):
        c = lax.axis_index("c")
        s = lax.axis_index("s")
        _fill(zbuf, ZR, 0.0)
        _fill(ones, 128, 1.0)
        _zero_my_slice(acc, zbuf, s)
        plsc.subcore_barrier()

        @pl.loop(0, ER // NC // NS // 16)
        def _(blk):
            row0 = c * (ER // NC) + s * (ER // NC // NS) + blk * 16
            pltpu.sync_copy(dst_hbm.at[pl.ds(row0, 16)], dstv)
            descs = [pltpu.async_copy(ones, acc.at[dstv.at[j]], sem, add=True)
                     for j in range(16)]
            for d in descs:
                d.wait()

        plsc.subcore_barrier()
        pltpu.sync_copy(acc.at[pl.ds(s * RPT, RPT)],
                        out_hbm.at[c, pl.ds(s * RPT, RPT)])

    return k(dst2)


def _sc_g_reshape(gw):
    @functools.partial(
        pl.kernel,
        out_type=jax.ShapeDtypeStruct((4 * NP, 16), jnp.float32),
        mesh=_mesh(),
        scratch_types=[
            pltpu.VMEM((NCH // 2, 128), jnp.float32),
            pltpu.VMEM((NCH, 16), jnp.float32),
            pltpu.VMEM((NCH, 16), jnp.float32),
            pltpu.VMEM((NCH, 16), jnp.float32),
            pltpu.VMEM((NCH, 16), jnp.float32),
        ],
        **_SC_PARAMS,
    )
    def k(gw_hbm, out_hbm, gwb, s0b, s1b, s2b, s3b):
        c = lax.axis_index("c")
        s = lax.axis_index("s")
        wid = s * NC + c
        slabs = (s0b, s1b, s2b, s3b)

        @pl.loop(0, NPW // NCH)
        def _(ch):
            n0 = wid * NPW + ch * NCH
            pltpu.sync_copy(gw_hbm.at[pl.ds(n0 // 2, NCH // 2)], gwb)

            @pl.loop(0, NCH // 2)
            def _(i):
                for par in range(2):
                    for cg in range(4):
                        slabs[cg][i * 2 + par, :] = (
                            gwb[i, pl.ds(par * 64 + 16 * cg, 16)])

            for cg in range(4):
                pltpu.sync_copy(slabs[cg],
                                out_hbm.at[pl.ds(cg * NP + n0, NCH)])

    return k(gw)


def _sc_s_reshape(S):
    @functools.partial(
        pl.kernel,
        out_type=jax.ShapeDtypeStruct((NP // 2, 128), jnp.float32),
        mesh=_mesh(),
        scratch_types=[
            pltpu.VMEM((NCH // 2, 128), jnp.float32),
            pltpu.VMEM((NCH, 16), jnp.float32),
            pltpu.VMEM((NCH, 16), jnp.float32),
            pltpu.VMEM((NCH, 16), jnp.float32),
            pltpu.VMEM((NCH, 16), jnp.float32),
        ],
        **_SC_PARAMS,
    )
    def k(s_hbm, out_hbm, spb, s0b, s1b, s2b, s3b):
        c = lax.axis_index("c")
        s = lax.axis_index("s")
        wid = s * NC + c
        slabs = (s0b, s1b, s2b, s3b)

        @pl.loop(0, NPW // NCH)
        def _(ch):
            n0 = wid * NPW + ch * NCH
            for cg in range(4):
                pltpu.sync_copy(s_hbm.at[cg, pl.ds(n0, NCH)], slabs[cg])

            @pl.loop(0, NCH // 2)
            def _(i):
                for par in range(2):
                    for cg in range(4):
                        spb[i, pl.ds(par * 64 + 16 * cg, 16)] = (
                            slabs[cg][i * 2 + par, :])

            pltpu.sync_copy(spb, out_hbm.at[pl.ds(n0 // 2, NCH // 2)])

    return k(S)


def _sc_segsum(src2, dst2, gflat):
    BR = 4
    NBLK = ROWS_T // BR
    HALF = NBLK // 2

    @functools.partial(
        pl.kernel,
        out_type=jax.ShapeDtypeStruct((4, NP, 16), jnp.float32),
        mesh=_mesh(),
        scratch_types=[
            pltpu.VMEM_SHARED((NP, 16), jnp.float32),
            pltpu.VMEM((ZR, 16), jnp.float32),
            pltpu.VMEM((BR, 128), jnp.int32), pltpu.VMEM((BR, 128), jnp.int32),
            pltpu.VMEM((BR, 128), jnp.int32), pltpu.VMEM((BR, 128), jnp.int32),
            pltpu.VMEM((BR, 128), jnp.int32), pltpu.VMEM((BR, 128), jnp.int32),
            pltpu.VMEM((BR, 128), jnp.int32), pltpu.VMEM((BR, 128), jnp.int32),
            pltpu.VMEM((BR, 128, 16), jnp.float32),
            pltpu.VMEM((BR, 128, 16), jnp.float32),
            pltpu.SemaphoreType.DMA, pltpu.SemaphoreType.DMA,
            pltpu.SemaphoreType.DMA, pltpu.SemaphoreType.DMA,
            pltpu.SemaphoreType.DMA,
        ],
        **_SC_PARAMS,
    )
    def k(src_hbm, dst_hbm, g_hbm, out_hbm, acc, zbuf,
          srcv0, srcv1, dstv0, dstv1, gidx0, gidx1, dstx0, dstx1,
          grow0, grow1, semi0, semi1, semg0, semg1, sems):
        c = lax.axis_index("c")
        s = lax.axis_index("s")
        srcv = (srcv0, srcv1)
        dstv = (dstv0, dstv1)
        gidx = (gidx0, gidx1)
        dstx = (dstx0, dstx1)
        grow = (grow0, grow1)
        semi = (semi0, semi1)
        semg = (semg0, semg1)
        _fill(zbuf, ZR, 0.0)

        def rowbase(b):
            return s * ROWS_T + b * BR

        def fire_idx(b, sl):
            r0 = rowbase(b)
            pltpu.async_copy(src_hbm.at[pl.ds(r0, BR)], srcv[sl], semi[sl])
            pltpu.async_copy(dst_hbm.at[pl.ds(r0, BR)], dstv[sl], semi[sl])

        def wait_g(sl):
            for j in range(BR):
                pltpu.make_async_copy(g_hbm.at[gidx[sl].at[j]],
                                      grow[sl].at[j], semg[sl]).wait()

        def scatter(sl):
            ds_ = [pltpu.async_copy(grow[sl].at[j], acc.at[dstx[sl].at[j]],
                                    sems, add=True)
                   for j in range(BR)]
            for d in ds_:
                d.wait()

        for cgl in range(2):
            cg = c * 2 + cgl

            def fire(b, sl, cg=cg):
                pltpu.make_async_copy(src_hbm.at[pl.ds(0, BR)], srcv[sl],
                                      semi[sl]).wait()
                pltpu.make_async_copy(dst_hbm.at[pl.ds(0, BR)], dstv[sl],
                                      semi[sl]).wait()
                for i in range(BR):
                    for m in range(8):
                        slc = pl.ds(m * 16, 16)
                        gidx[sl][i, slc] = srcv[sl][i, slc] + cg * NP
                        dstx[sl][i, slc] = dstv[sl][i, slc]
                for j in range(BR):
                    pltpu.async_copy(g_hbm.at[gidx[sl].at[j]], grow[sl].at[j],
                                     semg[sl])

            _zero_my_slice(acc, zbuf, s)
            plsc.subcore_barrier()
            fire_idx(0, 0)
            fire(0, 0)
            fire_idx(1, 1)

            @pl.loop(0, HALF)
            def _(kk):
                b0 = kk * 2
                fire(b0 + 1, 1)
                not_last = kk < HALF - 1

                @pl.when(not_last)
                def _():
                    fire_idx(b0 + 2, 0)

                wait_g(0)
                scatter(0)

                @pl.when(not_last)
                def _():
                    fire(b0 + 2, 0)
                    fire_idx(b0 + 3, 1)

                wait_g(1)
                scatter(1)

            plsc.subcore_barrier()
            pltpu.sync_copy(acc.at[pl.ds(s * RPT, RPT)],
                            out_hbm.at[cg, pl.ds(s * RPT, RPT)])

    return k(src2, dst2, gflat)


def _sc_mlp(src2, dst2, pb, qb, rb, wvec):
    BR = 2
    BE = BR * 128
    NBLK = ROWS_W // BR
    HALF = NBLK // 2

    @functools.partial(
        pl.kernel,
        out_type=jax.ShapeDtypeStruct((EP,), jnp.float32),
        mesh=_mesh(),
        scratch_types=[
            pltpu.VMEM((BR, 128), jnp.int32), pltpu.VMEM((BR, 128), jnp.int32),
            pltpu.VMEM((BR, 128), jnp.int32), pltpu.VMEM((BR, 128), jnp.int32),
            pltpu.VMEM((BR, 128), jnp.int32), pltpu.VMEM((BR, 128), jnp.int32),
            pltpu.VMEM((BR, 128), jnp.int32), pltpu.VMEM((BR, 128), jnp.int32),
            pltpu.VMEM((BE, HID), jnp.bfloat16),
            pltpu.VMEM((BE, HID), jnp.bfloat16),
            pltpu.VMEM((BE, HID), jnp.bfloat16),
            pltpu.VMEM((BE, HID), jnp.bfloat16),
            pltpu.VMEM((BE // 2, 128), jnp.float32),
            pltpu.VMEM((BE // 2, 128), jnp.float32),
            pltpu.VMEM((BE,), jnp.float32), pltpu.VMEM((BE,), jnp.float32),
            pltpu.VMEM((72,), jnp.float32),
            pltpu.SemaphoreType.DMA, pltpu.SemaphoreType.DMA,
            pltpu.SemaphoreType.DMA, pltpu.SemaphoreType.DMA,
        ],
        **_SC_PARAMS,
    )
    def k(src_hbm, dst_hbm, p_hbm, q_hbm, r_hbm, w_hbm, out_hbm,
          srcv0, srcv1, dstv0, dstv1, sidx0, sidx1, didx0, didx1,
          ps0, ps1, qd0, qd1, rv0, rv1,
          outv0, outv1, wv, semi0, semi1, semg0, semg1):
        c = lax.axis_index("c")
        s = lax.axis_index("s")
        wid = s * NC + c
        srcv = (srcv0, srcv1)
        dstv = (dstv0, dstv1)
        sidx = (sidx0, sidx1)
        didx = (didx0, didx1)
        psb = (ps0, ps1)
        qdb = (qd0, qd1)
        rv = (rv0, rv1)
        outv = (outv0, outv1)
        semi = (semi0, semi1)
        semg = (semg0, semg1)

        pltpu.sync_copy(w_hbm, wv)
        w_vc = [wv[pl.ds(16 * t, 16)] for t in range(4)]
        bm2s = wv[pl.ds(56, 16)][8]
        lane = lax.iota(jnp.int32, 16)

        def rowbase(b):
            return wid * ROWS_W + b * BR

        def fire_idx(b, sl):
            r0 = rowbase(b)
            pltpu.async_copy(src_hbm.at[pl.ds(r0, BR)], srcv[sl], semi[sl])
            pltpu.async_copy(dst_hbm.at[pl.ds(r0, BR)], dstv[sl], semi[sl])

        def fire(b, sl):
            pltpu.make_async_copy(src_hbm.at[pl.ds(0, BR)], srcv[sl],
                                  semi[sl]).wait()
            pltpu.make_async_copy(dst_hbm.at[pl.ds(0, BR)], dstv[sl],
                                  semi[sl]).wait()
            r0 = rowbase(b)
            for i in range(BR):
                for m in range(8):
                    slc = pl.ds(m * 16, 16)
                    sidx[sl][i, slc] = srcv[sl][i, slc]
                    didx[sl][i, slc] = dstv[sl][i, slc]
            for j in range(BR):
                pltpu.async_copy(p_hbm.at[sidx[sl].at[j]],
                                 psb[sl].at[pl.ds(j * 128, 128)], semg[sl])
            for j in range(BR):
                pltpu.async_copy(q_hbm.at[didx[sl].at[j]],
                                 qdb[sl].at[pl.ds(j * 128, 128)], semg[sl])
            pltpu.async_copy(r_hbm.at[pl.ds(r0 * 64, BE // 2)], rv[sl],
                             semg[sl])

        def wait_all(sl):
            pltpu.make_async_copy(p_hbm.at[pl.ds(0, BE)], psb[sl],
                                  semg[sl]).wait()
            pltpu.make_async_copy(q_hbm.at[pl.ds(0, BE)], qdb[sl],
                                  semg[sl]).wait()
            pltpu.make_async_copy(r_hbm.at[pl.ds(0, BE // 2)], rv[sl],
                                  semg[sl]).wait()

        def compute(b, sl):
            @pl.loop(0, BE // 16)
            def _(grp):
                res = jnp.zeros((16,), jnp.float32) + bm2s
                for l in range(16):
                    e = grp * 16 + l
                    row = grp * 8 + (l // 2)
                    off = (l % 2) * 64
                    acc = None
                    for t in range(2):
                        sp = pl.ds(32 * t, 32)
                        sv = psb[sl][e, sp] + qdb[sl][e, sp]
                        ae, ao = plsc.unpack(
                            sv, format=plsc.PackFormat.INTERLEAVED)
                        ze = jnp.maximum(
                            ae + rv[sl][row, pl.ds(off + 16 * 2 * t, 16)], 0.0)
                        zo = jnp.maximum(
                            ao + rv[sl][row, pl.ds(off + 16 * (2 * t + 1), 16)],
                            0.0)
                        pa = ze * w_vc[2 * t] + zo * w_vc[2 * t + 1]
                        acc = pa if acc is None else acc + pa
                    res = jnp.where(lane == l, res + jnp.sum(acc), res)
                outv[sl][pl.ds(grp * 16, 16)] = res

            pltpu.sync_copy(outv[sl], out_hbm.at[pl.ds(rowbase(b) * 128, BE)])

        fire_idx(0, 0)
        fire(0, 0)
        fire_idx(1, 1)

        @pl.loop(0, HALF)
        def _(kk):
            b0 = kk * 2
            fire(b0 + 1, 1)
            not_last = kk < HALF - 1

            @pl.when(not_last)
            def _():
                fire_idx(b0 + 2, 0)

            wait_all(0)
            compute(b0, 0)

            @pl.when(not_last)
            def _():
                fire(b0 + 2, 0)
                fire_idx(b0 + 3, 1)

            wait_all(1)
            compute(b0 + 1, 1)

    return k(src2, dst2, pb, qb, rb, wvec)


_PBLK = 512
_PGRID = NP // 2 // _PBLK


def _tc_prep(x8, dpair, W1):
    def body(x_ref, d_ref, w_ref, g_ref, dv_ref):
        w = w_ref[...]
        dL = lax.rsqrt(d_ref[:, 0:1] + 1.0)
        dR = lax.rsqrt(d_ref[:, 1:2] + 1.0)
        hL = jnp.dot(x_ref[:, 0:4], w, preferred_element_type=jnp.float32) * dL
        hR = jnp.dot(x_ref[:, 4:8], w, preferred_element_type=jnp.float32) * dR
        g_ref[...] = jnp.concatenate([hL, hR], axis=1)
        dv_ref[...] = jnp.concatenate([dL, dR], axis=1)

    return pl.pallas_call(
        body,
        grid=(_PGRID,),
        in_specs=[
            pl.BlockSpec((_PBLK, 8), lambda i: (i, 0)),
            pl.BlockSpec((_PBLK, 2), lambda i: (i, 0)),
            pl.BlockSpec((4, HID), lambda i: (0, 0)),
        ],
        out_specs=[
            pl.BlockSpec((_PBLK, 128), lambda i: (i, 0)),
            pl.BlockSpec((_PBLK, 2), lambda i: (i, 0)),
        ],
        out_shape=[
            jax.ShapeDtypeStruct((NP // 2, 128), jnp.float32),
            jax.ShapeDtypeStruct((NP // 2, 2), jnp.float32),
        ],
    )(x8, dpair, W1)


def _tc_combine(Sp, gw, dinv2, b_row, Wn):
    def body(s_ref, g_ref, d_ref, b_ref, w_ref, out_ref):
        w = w_ref[...]
        b = b_ref[...]
        dL = d_ref[:, 0:1]
        dR = d_ref[:, 1:2]
        hL = jnp.maximum(dL * (s_ref[:, 0:64] + g_ref[:, 0:64]) + b, 0.0)
        hR = jnp.maximum(dR * (s_ref[:, 64:128] + g_ref[:, 64:128]) + b, 0.0)
        oL = dL * jnp.dot(hL, w, preferred_element_type=jnp.float32)
        oR = dR * jnp.dot(hR, w, preferred_element_type=jnp.float32)
        out_ref[...] = jnp.concatenate([oL, oR], axis=1)

    return pl.pallas_call(
        body,
        grid=(_PGRID,),
        in_specs=[
            pl.BlockSpec((_PBLK, 128), lambda i: (i, 0)),
            pl.BlockSpec((_PBLK, 128), lambda i: (i, 0)),
            pl.BlockSpec((_PBLK, 2), lambda i: (i, 0)),
            pl.BlockSpec((1, HID), lambda i: (0, 0)),
            pl.BlockSpec((HID, HID), lambda i: (0, 0)),
        ],
        out_specs=pl.BlockSpec((_PBLK, 128), lambda i: (i, 0)),
        out_shape=jax.ShapeDtypeStruct((NP // 2, 128), jnp.float32),
    )(Sp, gw, dinv2, b_row, Wn)


def _tc_final_nodes(Sp, gw, dinv2, b_row, Wsrc, Wdst):
    def body(s_ref, g_ref, d_ref, b_ref, ws_ref, wd_ref,
             pl_ref, pr_ref, ql_ref, qr_ref):
        ws = ws_ref[...]
        wd = wd_ref[...]
        b = b_ref[...]
        dL = d_ref[:, 0:1]
        dR = d_ref[:, 1:2]
        hL = jnp.maximum(dL * (s_ref[:, 0:64] + g_ref[:, 0:64]) + b, 0.0)
        hR = jnp.maximum(dR * (s_ref[:, 64:128] + g_ref[:, 64:128]) + b, 0.0)
        pl_ref[...] = jnp.dot(hL, ws, preferred_element_type=jnp.float32
                              ).astype(jnp.bfloat16)
        pr_ref[...] = jnp.dot(hR, ws, preferred_element_type=jnp.float32
                              ).astype(jnp.bfloat16)
        ql_ref[...] = jnp.dot(hL, wd, preferred_element_type=jnp.float32
                              ).astype(jnp.bfloat16)
        qr_ref[...] = jnp.dot(hR, wd, preferred_element_type=jnp.float32
                              ).astype(jnp.bfloat16)

    return pl.pallas_call(
        body,
        grid=(_PGRID,),
        in_specs=[
            pl.BlockSpec((_PBLK, 128), lambda i: (i, 0)),
            pl.BlockSpec((_PBLK, 128), lambda i: (i, 0)),
            pl.BlockSpec((_PBLK, 2), lambda i: (i, 0)),
            pl.BlockSpec((1, HID), lambda i: (0, 0)),
            pl.BlockSpec((HID, HID), lambda i: (0, 0)),
            pl.BlockSpec((HID, HID), lambda i: (0, 0)),
        ],
        out_specs=[pl.BlockSpec((_PBLK, HID), lambda i: (i, 0))] * 4,
        out_shape=[jax.ShapeDtypeStruct((NP // 2, HID), jnp.bfloat16)] * 4,
    )(Sp, gw, dinv2, b_row, Wsrc, Wdst)


_EBLK = 1024


def _tc_edge_r(ea8, We, bm1_row):
    def body(ea_ref, w_ref, b_ref, r_ref):
        w = w_ref[...]
        b = b_ref[...]
        r0 = jnp.dot(ea_ref[:, 0:4], w, preferred_element_type=jnp.float32) + b
        r1 = jnp.dot(ea_ref[:, 4:8], w, preferred_element_type=jnp.float32) + b
        r_ref[...] = jnp.concatenate([r0, r1], axis=1)

    return pl.pallas_call(
        body,
        grid=(EP // 2 // _EBLK,),
        in_specs=[
            pl.BlockSpec((_EBLK, 8), lambda i: (i, 0)),
            pl.BlockSpec((4, HID), lambda i: (0, 0)),
            pl.BlockSpec((1, HID), lambda i: (0, 0)),
        ],
        out_specs=pl.BlockSpec((_EBLK, 128), lambda i: (i, 0)),
        out_shape=jax.ShapeDtypeStruct((EP // 2, 128), jnp.float32),
    )(ea8, We, bm1_row)


def kernel(x, edge_index, edge_attr, W1, b1, W2, b2, Wm1, bm1, Wm2, bm2):
    src = edge_index[0]
    dst = edge_index[1]
    src2 = jnp.full((EP,), N, jnp.int32).at[:E].set(src).reshape(ER, 128)
    dst2 = jnp.full((EP,), N, jnp.int32).at[:E].set(dst).reshape(ER, 128)
    ea8 = jnp.zeros((EP, 4), jnp.float32).at[:E].set(edge_attr).reshape(EP // 2, 8)
    x8 = jnp.zeros((NP, 4), jnp.float32).at[:N].set(x).reshape(NP // 2, 8)
    b1_row = b1.reshape(1, HID)
    b2_row = b2.reshape(1, HID)
    Wsrc = Wm1[0:HID]
    Wdst = Wm1[HID:2 * HID]
    perm = jnp.array(list(range(0, 32, 2)) + list(range(1, 32, 2))
                     + list(range(32, 64, 2)) + list(range(33, 64, 2)))
    Wep = Wm1[2 * HID:][:, perm]
    bm1p_row = bm1[perm].reshape(1, HID)
    wvec = (jnp.zeros((72,), jnp.float32)
            .at[0:HID].set(Wm2[perm, 0])
            .at[HID].set(bm2[0]))

    rb = _tc_edge_r(ea8, Wep, bm1p_row)
    deg_parts = _sc_deg(dst2)
    dpair = (deg_parts[0, :, 0] + deg_parts[1, :, 0]).reshape(NP // 2, 2)
    gw1, dinv2 = _tc_prep(x8, dpair, W1)
    gflat1 = _sc_g_reshape(gw1)
    S1p = _sc_s_reshape(_sc_segsum(src2, dst2, gflat1))
    gw2 = _tc_combine(S1p, gw1, dinv2, b1_row, W2)
    gflat2 = _sc_g_reshape(gw2)
    S2p = _sc_s_reshape(_sc_segsum(src2, dst2, gflat2))
    pL, pR, qL, qR = _tc_final_nodes(S2p, gw2, dinv2, b2_row, Wsrc, Wdst)
    p = jnp.stack([pL, pR], axis=1).reshape(NP, HID)
    q = jnp.stack([qL, qR], axis=1).reshape(NP, HID)
    logits = _sc_mlp(src2, dst2, p, q, rb, wvec)
    return logits[:E]


# speedup vs baseline: 14.4673x; 1.9482x over previous
import functools

import jax
import jax.numpy as jnp
from jax import lax
from jax.experimental import pallas as pl
from jax.experimental.pallas import tpu as pltpu
from jax.experimental.pallas import tpu_sc as plsc

N = 100000
E = 1600000
HID = 64

NC = 2
NS = 16
NW = NC * NS

NP = 100352
RPT = NP // NS
ZR = 98
NPW = NP // NW
NCH = 448

ER = 12800
EP = ER * 128
ROWS_W = ER // NW
ROWS_T = ER // NS

_SC_PARAMS = dict(
    compiler_params=pltpu.CompilerParams(use_tc_tiling_on_sc=False,
                                         needs_layout_passes=False),
)


def _mesh():
    return plsc.VectorSubcoreMesh(core_axis_name="c", subcore_axis_name="s",
                                  num_cores=NC, num_subcores=NS)


def _fill(ref, rows, val):
    v = jnp.full((16,), val, ref.dtype)

    @pl.loop(0, rows)
    def _(i):
        ref[i, :] = v


def _zero_my_slice(acc, zbuf, s):
    @pl.loop(0, 64)
    def _(b):
        pltpu.sync_copy(zbuf, acc.at[pl.ds(s * RPT + b * ZR, ZR)])


def _sc_deg(dst2):
    @functools.partial(
        pl.kernel,
        out_type=jax.ShapeDtypeStruct((NC, NP, 16), jnp.float32),
        mesh=_mesh(),
        scratch_types=[
            pltpu.VMEM_SHARED((NP, 16), jnp.float32),
            pltpu.VMEM((ZR, 16), jnp.float32),
            pltpu.VMEM((128, 16), jnp.float32),
            pltpu.VMEM((16, 128), jnp.int32),
            pltpu.SemaphoreType.DMA,
        ],
        **_SC_PARAMS,
    )
    def k(dst_hbm, out_hbm, acc, zbuf, ones, dstv, sem):
        c = lax.axis_index("c")
        s = lax.axis_index("s")
        _fill(zbuf, ZR, 0.0)
        _fill(ones, 128, 1.0)
        _zero_my_slice(acc, zbuf, s)
        plsc.subcore_barrier()

        @pl.loop(0, ER // NC // NS // 16)
        def _(blk):
            row0 = c * (ER // NC) + s * (ER // NC // NS) + blk * 16
            pltpu.sync_copy(dst_hbm.at[pl.ds(row0, 16)], dstv)
            descs = [pltpu.async_copy(ones, acc.at[dstv.at[j]], sem, add=True)
                     for j in range(16)]
            for d in descs:
                d.wait()

        plsc.subcore_barrier()
        pltpu.sync_copy(acc.at[pl.ds(s * RPT, RPT)],
                        out_hbm.at[c, pl.ds(s * RPT, RPT)])

    return k(dst2)


def _sc_g_reshape(gw):
    @functools.partial(
        pl.kernel,
        out_type=jax.ShapeDtypeStruct((4 * NP, 16), jnp.float32),
        mesh=_mesh(),
        scratch_types=[
            pltpu.VMEM((NCH // 2, 128), jnp.float32),
            pltpu.VMEM((NCH, 16), jnp.float32),
            pltpu.VMEM((NCH, 16), jnp.float32),
            pltpu.VMEM((NCH, 16), jnp.float32),
            pltpu.VMEM((NCH, 16), jnp.float32),
        ],
        **_SC_PARAMS,
    )
    def k(gw_hbm, out_hbm, gwb, s0b, s1b, s2b, s3b):
        c = lax.axis_index("c")
        s = lax.axis_index("s")
        wid = s * NC + c
        slabs = (s0b, s1b, s2b, s3b)

        @pl.loop(0, NPW // NCH)
        def _(ch):
            n0 = wid * NPW + ch * NCH
            pltpu.sync_copy(gw_hbm.at[pl.ds(n0 // 2, NCH // 2)], gwb)

            @pl.loop(0, NCH // 2)
            def _(i):
                for par in range(2):
                    for cg in range(4):
                        slabs[cg][i * 2 + par, :] = (
                            gwb[i, pl.ds(par * 64 + 16 * cg, 16)])

            for cg in range(4):
                pltpu.sync_copy(slabs[cg],
                                out_hbm.at[pl.ds(cg * NP + n0, NCH)])

    return k(gw)


def _sc_s_reshape(S):
    @functools.partial(
        pl.kernel,
        out_type=jax.ShapeDtypeStruct((NP // 2, 128), jnp.float32),
        mesh=_mesh(),
        scratch_types=[
            pltpu.VMEM((NCH // 2, 128), jnp.float32),
            pltpu.VMEM((NCH, 16), jnp.float32),
            pltpu.VMEM((NCH, 16), jnp.float32),
            pltpu.VMEM((NCH, 16), jnp.float32),
            pltpu.VMEM((NCH, 16), jnp.float32),
        ],
        **_SC_PARAMS,
    )
    def k(s_hbm, out_hbm, spb, s0b, s1b, s2b, s3b):
        c = lax.axis_index("c")
        s = lax.axis_index("s")
        wid = s * NC + c
        slabs = (s0b, s1b, s2b, s3b)

        @pl.loop(0, NPW // NCH)
        def _(ch):
            n0 = wid * NPW + ch * NCH
            for cg in range(4):
                pltpu.sync_copy(s_hbm.at[cg, pl.ds(n0, NCH)], slabs[cg])

            @pl.loop(0, NCH // 2)
            def _(i):
                for par in range(2):
                    for cg in range(4):
                        spb[i, pl.ds(par * 64 + 16 * cg, 16)] = (
                            slabs[cg][i * 2 + par, :])

            pltpu.sync_copy(spb, out_hbm.at[pl.ds(n0 // 2, NCH // 2)])

    return k(S)


def _sc_segsum(src2, dst2, gflat):
    BR = 4
    NBLK = ROWS_T // BR
    HALF = NBLK // 2

    @functools.partial(
        pl.kernel,
        out_type=jax.ShapeDtypeStruct((4, NP, 16), jnp.float32),
        mesh=_mesh(),
        scratch_types=[
            pltpu.VMEM_SHARED((NP, 16), jnp.float32),
            pltpu.VMEM((ZR, 16), jnp.float32),
            pltpu.VMEM((BR, 128), jnp.int32), pltpu.VMEM((BR, 128), jnp.int32),
            pltpu.VMEM((BR, 128), jnp.int32), pltpu.VMEM((BR, 128), jnp.int32),
            pltpu.VMEM((BR, 128), jnp.int32), pltpu.VMEM((BR, 128), jnp.int32),
            pltpu.VMEM((BR, 128), jnp.int32), pltpu.VMEM((BR, 128), jnp.int32),
            pltpu.VMEM((BR, 128, 16), jnp.float32),
            pltpu.VMEM((BR, 128, 16), jnp.float32),
            pltpu.SemaphoreType.DMA, pltpu.SemaphoreType.DMA,
            pltpu.SemaphoreType.DMA, pltpu.SemaphoreType.DMA,
            pltpu.SemaphoreType.DMA,
        ],
        **_SC_PARAMS,
    )
    def k(src_hbm, dst_hbm, g_hbm, out_hbm, acc, zbuf,
          srcv0, srcv1, dstv0, dstv1, gidx0, gidx1, dstx0, dstx1,
          grow0, grow1, semi0, semi1, semg0, semg1, sems):
        c = lax.axis_index("c")
        s = lax.axis_index("s")
        srcv = (srcv0, srcv1)
        dstv = (dstv0, dstv1)
        gidx = (gidx0, gidx1)
        dstx = (dstx0, dstx1)
        grow = (grow0, grow1)
        semi = (semi0, semi1)
        semg = (semg0, semg1)
        _fill(zbuf, ZR, 0.0)

        def rowbase(b):
            return s * ROWS_T + b * BR

        def fire_idx(b, sl):
            r0 = rowbase(b)
            pltpu.async_copy(src_hbm.at[pl.ds(r0, BR)], srcv[sl], semi[sl])
            pltpu.async_copy(dst_hbm.at[pl.ds(r0, BR)], dstv[sl], semi[sl])

        def wait_g(sl):
            for j in range(BR):
                pltpu.make_async_copy(g_hbm.at[gidx[sl].at[j]],
                                      grow[sl].at[j], semg[sl]).wait()

        def scatter(sl):
            ds_ = [pltpu.async_copy(grow[sl].at[j], acc.at[dstx[sl].at[j]],
                                    sems, add=True)
                   for j in range(BR)]
            for d in ds_:
                d.wait()

        for cgl in range(2):
            cg = c * 2 + cgl

            def fire(b, sl, cg=cg):
                pltpu.make_async_copy(src_hbm.at[pl.ds(0, BR)], srcv[sl],
                                      semi[sl]).wait()
                pltpu.make_async_copy(dst_hbm.at[pl.ds(0, BR)], dstv[sl],
                                      semi[sl]).wait()
                for i in range(BR):
                    for m in range(8):
                        slc = pl.ds(m * 16, 16)
                        gidx[sl][i, slc] = srcv[sl][i, slc] + cg * NP
                        dstx[sl][i, slc] = dstv[sl][i, slc]
                for j in range(BR):
                    pltpu.async_copy(g_hbm.at[gidx[sl].at[j]], grow[sl].at[j],
                                     semg[sl])

            _zero_my_slice(acc, zbuf, s)
            plsc.subcore_barrier()
            fire_idx(0, 0)
            fire(0, 0)
            fire_idx(1, 1)

            @pl.loop(0, HALF)
            def _(kk):
                b0 = kk * 2
                fire(b0 + 1, 1)
                not_last = kk < HALF - 1

                @pl.when(not_last)
                def _():
                    fire_idx(b0 + 2, 0)

                wait_g(0)
                scatter(0)

                @pl.when(not_last)
                def _():
                    fire(b0 + 2, 0)
                    fire_idx(b0 + 3, 1)

                wait_g(1)
                scatter(1)

            plsc.subcore_barrier()
            pltpu.sync_copy(acc.at[pl.ds(s * RPT, RPT)],
                            out_hbm.at[cg, pl.ds(s * RPT, RPT)])

    return k(src2, dst2, gflat)


def _sc_mlp(src2, dst2, pb, qb, eaT, wvec):
    BR = 4
    BE = BR * 128
    NBLK = ROWS_W // BR
    HALF = NBLK // 2

    @functools.partial(
        pl.kernel,
        out_type=jax.ShapeDtypeStruct((EP,), jnp.float32),
        mesh=_mesh(),
        scratch_types=[
            pltpu.VMEM((BR, 128), jnp.int32), pltpu.VMEM((BR, 128), jnp.int32),
            pltpu.VMEM((BR, 128), jnp.int32), pltpu.VMEM((BR, 128), jnp.int32),
            pltpu.VMEM((BR, 128), jnp.int32), pltpu.VMEM((BR, 128), jnp.int32),
            pltpu.VMEM((BR, 128), jnp.int32), pltpu.VMEM((BR, 128), jnp.int32),
            pltpu.VMEM((BE, HID), jnp.bfloat16),
            pltpu.VMEM((BE, HID), jnp.bfloat16),
            pltpu.VMEM((BE, HID), jnp.bfloat16),
            pltpu.VMEM((BE, HID), jnp.bfloat16),
            pltpu.VMEM((4, BR, 128), jnp.float32),
            pltpu.VMEM((4, BR, 128), jnp.float32),
            pltpu.VMEM((BE,), jnp.float32), pltpu.VMEM((BE,), jnp.float32),
            pltpu.VMEM((392,), jnp.float32),
            pltpu.SemaphoreType.DMA, pltpu.SemaphoreType.DMA,
            pltpu.SemaphoreType.DMA, pltpu.SemaphoreType.DMA,
        ],
        **_SC_PARAMS,
    )
    def k(src_hbm, dst_hbm, p_hbm, q_hbm, ea_hbm, w_hbm, out_hbm,
          srcv0, srcv1, dstv0, dstv1, sidx0, sidx1, didx0, didx1,
          ps0, ps1, qd0, qd1, ea0, ea1,
          outv0, outv1, wv, semi0, semi1, semg0, semg1):
        c = lax.axis_index("c")
        s = lax.axis_index("s")
        wid = s * NC + c
        srcv = (srcv0, srcv1)
        dstv = (dstv0, dstv1)
        sidx = (sidx0, sidx1)
        didx = (didx0, didx1)
        psb = (ps0, ps1)
        qdb = (qd0, qd1)
        eab = (ea0, ea1)
        outv = (outv0, outv1)
        semi = (semi0, semi1)
        semg = (semg0, semg1)

        pltpu.sync_copy(w_hbm, wv)
        w_vc = [wv[pl.ds(16 * t, 16)] for t in range(4)]
        bm1_vc = [wv[pl.ds(64 + 16 * t, 16)] for t in range(4)]
        wek = [[wv[pl.ds(128 + 64 * kk + 16 * t, 16)] for t in range(4)]
               for kk in range(4)]
        bm2s = wv[pl.ds(376, 16)][8]
        lane = lax.iota(jnp.int32, 16)

        def rowbase(b):
            return wid * ROWS_W + b * BR

        def fire_idx(b, sl):
            r0 = rowbase(b)
            pltpu.async_copy(src_hbm.at[pl.ds(r0, BR)], srcv[sl], semi[sl])
            pltpu.async_copy(dst_hbm.at[pl.ds(r0, BR)], dstv[sl], semi[sl])

        def fire(b, sl):
            pltpu.make_async_copy(src_hbm.at[pl.ds(0, BR)], srcv[sl],
                                  semi[sl]).wait()
            pltpu.make_async_copy(dst_hbm.at[pl.ds(0, BR)], dstv[sl],
                                  semi[sl]).wait()
            r0 = rowbase(b)
            for i in range(BR):
                for m in range(8):
                    slc = pl.ds(m * 16, 16)
                    sidx[sl][i, slc] = srcv[sl][i, slc]
                    didx[sl][i, slc] = dstv[sl][i, slc]
            for j in range(BR):
                pltpu.async_copy(p_hbm.at[sidx[sl].at[j]],
                                 psb[sl].at[pl.ds(j * 128, 128)], semg[sl])
            for j in range(BR):
                pltpu.async_copy(q_hbm.at[didx[sl].at[j]],
                                 qdb[sl].at[pl.ds(j * 128, 128)], semg[sl])
            for kk in range(4):
                pltpu.async_copy(ea_hbm.at[kk, pl.ds(r0, BR)],
                                 eab[sl].at[kk], semg[sl])

        def wait_all(sl):
            pltpu.make_async_copy(p_hbm.at[pl.ds(0, BE)], psb[sl],
                                  semg[sl]).wait()
            pltpu.make_async_copy(q_hbm.at[pl.ds(0, BE)], qdb[sl],
                                  semg[sl]).wait()
            for kk in range(4):
                pltpu.make_async_copy(ea_hbm.at[kk, pl.ds(0, BR)],
                                      eab[sl].at[kk], semg[sl]).wait()

        def compute(b, sl):
            @pl.loop(0, BE // 16)
            def _(grp):
                res = jnp.zeros((16,), jnp.float32) + bm2s
                erow = grp // 8
                eoff = (grp % 8) * 16
                eav = [eab[sl][kk, erow, pl.ds(eoff, 16)] for kk in range(4)]
                for l in range(16):
                    e = grp * 16 + l
                    sks = [eav[kk][l] for kk in range(4)]
                    rvc = []
                    for t in range(4):
                        rv = bm1_vc[t]
                        for kk in range(4):
                            rv = rv + sks[kk] * wek[kk][t]
                        rvc.append(rv)
                    acc = None
                    for t in range(2):
                        sp = pl.ds(32 * t, 32)
                        sv = psb[sl][e, sp] + qdb[sl][e, sp]
                        ae, ao = plsc.unpack(
                            sv, format=plsc.PackFormat.INTERLEAVED)
                        ze = jnp.maximum(ae + rvc[2 * t], 0.0)
                        zo = jnp.maximum(ao + rvc[2 * t + 1], 0.0)
                        pa = ze * w_vc[2 * t] + zo * w_vc[2 * t + 1]
                        acc = pa if acc is None else acc + pa
                    res = jnp.where(lane == l, res + jnp.sum(acc), res)
                outv[sl][pl.ds(grp * 16, 16)] = res

            pltpu.sync_copy(outv[sl], out_hbm.at[pl.ds(rowbase(b) * 128, BE)])

        fire_idx(0, 0)
        fire(0, 0)
        fire_idx(1, 1)

        @pl.loop(0, HALF)
        def _(kk):
            b0 = kk * 2
            fire(b0 + 1, 1)
            not_last = kk < HALF - 1

            @pl.when(not_last)
            def _():
                fire_idx(b0 + 2, 0)

            wait_all(0)
            compute(b0, 0)

            @pl.when(not_last)
            def _():
                fire(b0 + 2, 0)
                fire_idx(b0 + 3, 1)

            wait_all(1)
            compute(b0 + 1, 1)

    return k(src2, dst2, pb, qb, eaT, wvec)


_PBLK = 512
_PGRID = NP // 2 // _PBLK


def _tc_prep(x8, dpair, W1):
    def body(x_ref, d_ref, w_ref, g_ref, dv_ref):
        w = w_ref[...]
        dL = lax.rsqrt(d_ref[:, 0:1] + 1.0)
        dR = lax.rsqrt(d_ref[:, 1:2] + 1.0)
        hL = jnp.dot(x_ref[:, 0:4], w, preferred_element_type=jnp.float32) * dL
        hR = jnp.dot(x_ref[:, 4:8], w, preferred_element_type=jnp.float32) * dR
        g_ref[...] = jnp.concatenate([hL, hR], axis=1)
        dv_ref[...] = jnp.concatenate([dL, dR], axis=1)

    return pl.pallas_call(
        body,
        grid=(_PGRID,),
        in_specs=[
            pl.BlockSpec((_PBLK, 8), lambda i: (i, 0)),
            pl.BlockSpec((_PBLK, 2), lambda i: (i, 0)),
            pl.BlockSpec((4, HID), lambda i: (0, 0)),
        ],
        out_specs=[
            pl.BlockSpec((_PBLK, 128), lambda i: (i, 0)),
            pl.BlockSpec((_PBLK, 2), lambda i: (i, 0)),
        ],
        out_shape=[
            jax.ShapeDtypeStruct((NP // 2, 128), jnp.float32),
            jax.ShapeDtypeStruct((NP // 2, 2), jnp.float32),
        ],
    )(x8, dpair, W1)


def _tc_combine(Sp, gw, dinv2, b_row, Wn):
    def body(s_ref, g_ref, d_ref, b_ref, w_ref, out_ref):
        w = w_ref[...]
        b = b_ref[...]
        dL = d_ref[:, 0:1]
        dR = d_ref[:, 1:2]
        hL = jnp.maximum(dL * (s_ref[:, 0:64] + g_ref[:, 0:64]) + b, 0.0)
        hR = jnp.maximum(dR * (s_ref[:, 64:128] + g_ref[:, 64:128]) + b, 0.0)
        oL = dL * jnp.dot(hL, w, preferred_element_type=jnp.float32)
        oR = dR * jnp.dot(hR, w, preferred_element_type=jnp.float32)
        out_ref[...] = jnp.concatenate([oL, oR], axis=1)

    return pl.pallas_call(
        body,
        grid=(_PGRID,),
        in_specs=[
            pl.BlockSpec((_PBLK, 128), lambda i: (i, 0)),
            pl.BlockSpec((_PBLK, 128), lambda i: (i, 0)),
            pl.BlockSpec((_PBLK, 2), lambda i: (i, 0)),
            pl.BlockSpec((1, HID), lambda i: (0, 0)),
            pl.BlockSpec((HID, HID), lambda i: (0, 0)),
        ],
        out_specs=pl.BlockSpec((_PBLK, 128), lambda i: (i, 0)),
        out_shape=jax.ShapeDtypeStruct((NP // 2, 128), jnp.float32),
    )(Sp, gw, dinv2, b_row, Wn)


def _tc_final_nodes(Sp, gw, dinv2, b_row, Wsrc, Wdst):
    def body(s_ref, g_ref, d_ref, b_ref, ws_ref, wd_ref,
             pl_ref, pr_ref, ql_ref, qr_ref):
        ws = ws_ref[...]
        wd = wd_ref[...]
        b = b_ref[...]
        dL = d_ref[:, 0:1]
        dR = d_ref[:, 1:2]
        hL = jnp.maximum(dL * (s_ref[:, 0:64] + g_ref[:, 0:64]) + b, 0.0)
        hR = jnp.maximum(dR * (s_ref[:, 64:128] + g_ref[:, 64:128]) + b, 0.0)
        pl_ref[...] = jnp.dot(hL, ws, preferred_element_type=jnp.float32
                              ).astype(jnp.bfloat16)
        pr_ref[...] = jnp.dot(hR, ws, preferred_element_type=jnp.float32
                              ).astype(jnp.bfloat16)
        ql_ref[...] = jnp.dot(hL, wd, preferred_element_type=jnp.float32
                              ).astype(jnp.bfloat16)
        qr_ref[...] = jnp.dot(hR, wd, preferred_element_type=jnp.float32
                              ).astype(jnp.bfloat16)

    return pl.pallas_call(
        body,
        grid=(_PGRID,),
        in_specs=[
            pl.BlockSpec((_PBLK, 128), lambda i: (i, 0)),
            pl.BlockSpec((_PBLK, 128), lambda i: (i, 0)),
            pl.BlockSpec((_PBLK, 2), lambda i: (i, 0)),
            pl.BlockSpec((1, HID), lambda i: (0, 0)),
            pl.BlockSpec((HID, HID), lambda i: (0, 0)),
            pl.BlockSpec((HID, HID), lambda i: (0, 0)),
        ],
        out_specs=[pl.BlockSpec((_PBLK, HID), lambda i: (i, 0))] * 4,
        out_shape=[jax.ShapeDtypeStruct((NP // 2, HID), jnp.bfloat16)] * 4,
    )(Sp, gw, dinv2, b_row, Wsrc, Wdst)


def kernel(x, edge_index, edge_attr, W1, b1, W2, b2, Wm1, bm1, Wm2, bm2):
    src = edge_index[0]
    dst = edge_index[1]
    src2 = jnp.full((EP,), N, jnp.int32).at[:E].set(src).reshape(ER, 128)
    dst2 = jnp.full((EP,), N, jnp.int32).at[:E].set(dst).reshape(ER, 128)
    eaT = (jnp.zeros((EP, 4), jnp.float32).at[:E].set(edge_attr)
           .T.reshape(4, ER, 128))
    x8 = jnp.zeros((NP, 4), jnp.float32).at[:N].set(x).reshape(NP // 2, 8)
    b1_row = b1.reshape(1, HID)
    b2_row = b2.reshape(1, HID)
    Wsrc = Wm1[0:HID]
    Wdst = Wm1[HID:2 * HID]
    perm = jnp.array(list(range(0, 32, 2)) + list(range(1, 32, 2))
                     + list(range(32, 64, 2)) + list(range(33, 64, 2)))
    Wep = Wm1[2 * HID:][:, perm]
    wvec = (jnp.zeros((392,), jnp.float32)
            .at[0:HID].set(Wm2[perm, 0])
            .at[HID:2 * HID].set(bm1[perm])
            .at[2 * HID:2 * HID + 256].set(Wep.reshape(-1))
            .at[384].set(bm2[0]))

    deg_parts = _sc_deg(dst2)
    dpair = (deg_parts[0, :, 0] + deg_parts[1, :, 0]).reshape(NP // 2, 2)
    gw1, dinv2 = _tc_prep(x8, dpair, W1)
    gflat1 = _sc_g_reshape(gw1)
    S1p = _sc_s_reshape(_sc_segsum(src2, dst2, gflat1))
    gw2 = _tc_combine(S1p, gw1, dinv2, b1_row, W2)
    gflat2 = _sc_g_reshape(gw2)
    S2p = _sc_s_reshape(_sc_segsum(src2, dst2, gflat2))
    pL, pR, qL, qR = _tc_final_nodes(S2p, gw2, dinv2, b2_row, Wsrc, Wdst)
    p = jnp.stack([pL, pR], axis=1).reshape(NP, HID)
    q = jnp.stack([qL, qR], axis=1).reshape(NP, HID)
    logits = _sc_mlp(src2, dst2, p, q, eaT, wvec)
    return logits[:E]


# speedup vs baseline: 14.5923x; 1.0086x over previous
import functools

import jax
import jax.numpy as jnp
from jax import lax
from jax.experimental import pallas as pl
from jax.experimental.pallas import tpu as pltpu
from jax.experimental.pallas import tpu_sc as plsc

N = 100000
E = 1600000
HID = 64

NC = 2
NS = 16
NW = NC * NS

NP = 100352
RPT = NP // NS
ZR = 98
NPW = NP // NW
NCH = 448

ER = 12800
EP = ER * 128
ROWS_W = ER // NW
ROWS_T = ER // NS

_SC_PARAMS = dict(
    compiler_params=pltpu.CompilerParams(use_tc_tiling_on_sc=False,
                                         needs_layout_passes=False),
)


def _mesh():
    return plsc.VectorSubcoreMesh(core_axis_name="c", subcore_axis_name="s",
                                  num_cores=NC, num_subcores=NS)


def _fill(ref, rows, val):
    v = jnp.full((16,), val, ref.dtype)

    @pl.loop(0, rows)
    def _(i):
        ref[i, :] = v


def _zero_my_slice(acc, zbuf, s):
    @pl.loop(0, 64)
    def _(b):
        pltpu.sync_copy(zbuf, acc.at[pl.ds(s * RPT + b * ZR, ZR)])


def _sc_deg(dst2):
    @functools.partial(
        pl.kernel,
        out_type=jax.ShapeDtypeStruct((NC, NP, 16), jnp.float32),
        mesh=_mesh(),
        scratch_types=[
            pltpu.VMEM_SHARED((NP, 16), jnp.float32),
            pltpu.VMEM((ZR, 16), jnp.float32),
            pltpu.VMEM((128, 16), jnp.float32),
            pltpu.VMEM((16, 128), jnp.int32),
            pltpu.SemaphoreType.DMA,
        ],
        **_SC_PARAMS,
    )
    def k(dst_hbm, out_hbm, acc, zbuf, ones, dstv, sem):
        c = lax.axis_index("c")
        s = lax.axis_index("s")
        _fill(zbuf, ZR, 0.0)
        _fill(ones, 128, 1.0)
        _zero_my_slice(acc, zbuf, s)
        plsc.subcore_barrier()

        @pl.loop(0, ER // NC // NS // 16)
        def _(blk):
            row0 = c * (ER // NC) + s * (ER // NC // NS) + blk * 16
            pltpu.sync_copy(dst_hbm.at[pl.ds(row0, 16)], dstv)
            descs = [pltpu.async_copy(ones, acc.at[dstv.at[j]], sem, add=True)
                     for j in range(16)]
            for d in descs:
                d.wait()

        plsc.subcore_barrier()
        pltpu.sync_copy(acc.at[pl.ds(s * RPT, RPT)],
                        out_hbm.at[c, pl.ds(s * RPT, RPT)])

    return k(dst2)


def _sc_g_reshape(gw):
    @functools.partial(
        pl.kernel,
        out_type=jax.ShapeDtypeStruct((4 * NP, 16), jnp.float32),
        mesh=_mesh(),
        scratch_types=[
            pltpu.VMEM((NCH // 2, 128), jnp.float32),
            pltpu.VMEM((NCH, 16), jnp.float32),
            pltpu.VMEM((NCH, 16), jnp.float32),
            pltpu.VMEM((NCH, 16), jnp.float32),
            pltpu.VMEM((NCH, 16), jnp.float32),
        ],
        **_SC_PARAMS,
    )
    def k(gw_hbm, out_hbm, gwb, s0b, s1b, s2b, s3b):
        c = lax.axis_index("c")
        s = lax.axis_index("s")
        wid = s * NC + c
        slabs = (s0b, s1b, s2b, s3b)

        @pl.loop(0, NPW // NCH)
        def _(ch):
            n0 = wid * NPW + ch * NCH
            pltpu.sync_copy(gw_hbm.at[pl.ds(n0 // 2, NCH // 2)], gwb)

            @pl.loop(0, NCH // 2)
            def _(i):
                for par in range(2):
                    for cg in range(4):
                        slabs[cg][i * 2 + par, :] = (
                            gwb[i, pl.ds(par * 64 + 16 * cg, 16)])

            for cg in range(4):
                pltpu.sync_copy(slabs[cg],
                                out_hbm.at[pl.ds(cg * NP + n0, NCH)])

    return k(gw)


def _sc_s_reshape(S):
    @functools.partial(
        pl.kernel,
        out_type=jax.ShapeDtypeStruct((NP // 2, 128), jnp.float32),
        mesh=_mesh(),
        scratch_types=[
            pltpu.VMEM((NCH // 2, 128), jnp.float32),
            pltpu.VMEM((NCH, 16), jnp.float32),
            pltpu.VMEM((NCH, 16), jnp.float32),
            pltpu.VMEM((NCH, 16), jnp.float32),
            pltpu.VMEM((NCH, 16), jnp.float32),
        ],
        **_SC_PARAMS,
    )
    def k(s_hbm, out_hbm, spb, s0b, s1b, s2b, s3b):
        c = lax.axis_index("c")
        s = lax.axis_index("s")
        wid = s * NC + c
        slabs = (s0b, s1b, s2b, s3b)

        @pl.loop(0, NPW // NCH)
        def _(ch):
            n0 = wid * NPW + ch * NCH
            for cg in range(4):
                pltpu.sync_copy(s_hbm.at[cg, pl.ds(n0, NCH)], slabs[cg])

            @pl.loop(0, NCH // 2)
            def _(i):
                for par in range(2):
                    for cg in range(4):
                        spb[i, pl.ds(par * 64 + 16 * cg, 16)] = (
                            slabs[cg][i * 2 + par, :])

            pltpu.sync_copy(spb, out_hbm.at[pl.ds(n0 // 2, NCH // 2)])

    return k(S)


def _sc_segsum(src2, dst2, gflat):
    BR = 5
    NBLK = ROWS_T // BR
    HALF = NBLK // 2

    @functools.partial(
        pl.kernel,
        out_type=jax.ShapeDtypeStruct((4, NP, 16), jnp.float32),
        mesh=_mesh(),
        scratch_types=[
            pltpu.VMEM_SHARED((NP, 16), jnp.float32),
            pltpu.VMEM((ZR, 16), jnp.float32),
            pltpu.VMEM((BR, 128), jnp.int32), pltpu.VMEM((BR, 128), jnp.int32),
            pltpu.VMEM((BR, 128), jnp.int32), pltpu.VMEM((BR, 128), jnp.int32),
            pltpu.VMEM((BR, 128), jnp.int32), pltpu.VMEM((BR, 128), jnp.int32),
            pltpu.VMEM((BR, 128), jnp.int32), pltpu.VMEM((BR, 128), jnp.int32),
            pltpu.VMEM((BR, 128, 16), jnp.float32),
            pltpu.VMEM((BR, 128, 16), jnp.float32),
            pltpu.SemaphoreType.DMA, pltpu.SemaphoreType.DMA,
            pltpu.SemaphoreType.DMA, pltpu.SemaphoreType.DMA,
            pltpu.SemaphoreType.DMA,
        ],
        **_SC_PARAMS,
    )
    def k(src_hbm, dst_hbm, g_hbm, out_hbm, acc, zbuf,
          srcv0, srcv1, dstv0, dstv1, gidx0, gidx1, dstx0, dstx1,
          grow0, grow1, semi0, semi1, semg0, semg1, sems):
        c = lax.axis_index("c")
        s = lax.axis_index("s")
        srcv = (srcv0, srcv1)
        dstv = (dstv0, dstv1)
        gidx = (gidx0, gidx1)
        dstx = (dstx0, dstx1)
        grow = (grow0, grow1)
        semi = (semi0, semi1)
        semg = (semg0, semg1)
        _fill(zbuf, ZR, 0.0)

        def rowbase(b):
            return s * ROWS_T + b * BR

        def fire_idx(b, sl):
            r0 = rowbase(b)
            pltpu.async_copy(src_hbm.at[pl.ds(r0, BR)], srcv[sl], semi[sl])
            pltpu.async_copy(dst_hbm.at[pl.ds(r0, BR)], dstv[sl], semi[sl])

        def wait_g(sl):
            for j in range(BR):
                pltpu.make_async_copy(g_hbm.at[gidx[sl].at[j]],
                                      grow[sl].at[j], semg[sl]).wait()

        def scatter(sl):
            ds_ = [pltpu.async_copy(grow[sl].at[j], acc.at[dstx[sl].at[j]],
                                    sems, add=True)
                   for j in range(BR)]
            for d in ds_:
                d.wait()

        for cgl in range(2):
            cg = c * 2 + cgl

            def fire(b, sl, cg=cg):
                pltpu.make_async_copy(src_hbm.at[pl.ds(0, BR)], srcv[sl],
                                      semi[sl]).wait()
                pltpu.make_async_copy(dst_hbm.at[pl.ds(0, BR)], dstv[sl],
                                      semi[sl]).wait()
                for i in range(BR):
                    for m in range(8):
                        slc = pl.ds(m * 16, 16)
                        gidx[sl][i, slc] = srcv[sl][i, slc] + cg * NP
                        dstx[sl][i, slc] = dstv[sl][i, slc]
                for j in range(BR):
                    pltpu.async_copy(g_hbm.at[gidx[sl].at[j]], grow[sl].at[j],
                                     semg[sl])

            _zero_my_slice(acc, zbuf, s)
            plsc.subcore_barrier()
            fire_idx(0, 0)
            fire(0, 0)
            fire_idx(1, 1)

            @pl.loop(0, HALF)
            def _(kk):
                b0 = kk * 2
                fire(b0 + 1, 1)
                not_last = kk < HALF - 1

                @pl.when(not_last)
                def _():
                    fire_idx(b0 + 2, 0)

                wait_g(0)
                scatter(0)

                @pl.when(not_last)
                def _():
                    fire(b0 + 2, 0)
                    fire_idx(b0 + 3, 1)

                wait_g(1)
                scatter(1)

            plsc.subcore_barrier()
            pltpu.sync_copy(acc.at[pl.ds(s * RPT, RPT)],
                            out_hbm.at[cg, pl.ds(s * RPT, RPT)])

    return k(src2, dst2, gflat)


def _sc_mlp(src2, dst2, pb, qb, eaT, wvec):
    BR = 4
    BE = BR * 128
    NBLK = ROWS_W // BR
    HALF = NBLK // 2

    @functools.partial(
        pl.kernel,
        out_type=jax.ShapeDtypeStruct((EP,), jnp.float32),
        mesh=_mesh(),
        scratch_types=[
            pltpu.VMEM((BR, 128), jnp.int32), pltpu.VMEM((BR, 128), jnp.int32),
            pltpu.VMEM((BR, 128), jnp.int32), pltpu.VMEM((BR, 128), jnp.int32),
            pltpu.VMEM((BR, 128), jnp.int32), pltpu.VMEM((BR, 128), jnp.int32),
            pltpu.VMEM((BR, 128), jnp.int32), pltpu.VMEM((BR, 128), jnp.int32),
            pltpu.VMEM((BE, HID), jnp.bfloat16),
            pltpu.VMEM((BE, HID), jnp.bfloat16),
            pltpu.VMEM((BE, HID), jnp.bfloat16),
            pltpu.VMEM((BE, HID), jnp.bfloat16),
            pltpu.VMEM((4, BR, 128), jnp.float32),
            pltpu.VMEM((4, BR, 128), jnp.float32),
            pltpu.VMEM((BE,), jnp.float32), pltpu.VMEM((BE,), jnp.float32),
            pltpu.VMEM((392,), jnp.float32),
            pltpu.SemaphoreType.DMA, pltpu.SemaphoreType.DMA,
            pltpu.SemaphoreType.DMA, pltpu.SemaphoreType.DMA,
        ],
        **_SC_PARAMS,
    )
    def k(src_hbm, dst_hbm, p_hbm, q_hbm, ea_hbm, w_hbm, out_hbm,
          srcv0, srcv1, dstv0, dstv1, sidx0, sidx1, didx0, didx1,
          ps0, ps1, qd0, qd1, ea0, ea1,
          outv0, outv1, wv, semi0, semi1, semg0, semg1):
        c = lax.axis_index("c")
        s = lax.axis_index("s")
        wid = s * NC + c
        srcv = (srcv0, srcv1)
        dstv = (dstv0, dstv1)
        sidx = (sidx0, sidx1)
        didx = (didx0, didx1)
        psb = (ps0, ps1)
        qdb = (qd0, qd1)
        eab = (ea0, ea1)
        outv = (outv0, outv1)
        semi = (semi0, semi1)
        semg = (semg0, semg1)

        pltpu.sync_copy(w_hbm, wv)
        w_vc = [wv[pl.ds(16 * t, 16)] for t in range(4)]
        bm1_vc = [wv[pl.ds(64 + 16 * t, 16)] for t in range(4)]
        wek = [[wv[pl.ds(128 + 64 * kk + 16 * t, 16)] for t in range(4)]
               for kk in range(4)]
        bm2s = wv[pl.ds(376, 16)][8]
        lane = lax.iota(jnp.int32, 16)

        def rowbase(b):
            return wid * ROWS_W + b * BR

        def fire_idx(b, sl):
            r0 = rowbase(b)
            pltpu.async_copy(src_hbm.at[pl.ds(r0, BR)], srcv[sl], semi[sl])
            pltpu.async_copy(dst_hbm.at[pl.ds(r0, BR)], dstv[sl], semi[sl])

        def fire(b, sl):
            pltpu.make_async_copy(src_hbm.at[pl.ds(0, BR)], srcv[sl],
                                  semi[sl]).wait()
            pltpu.make_async_copy(dst_hbm.at[pl.ds(0, BR)], dstv[sl],
                                  semi[sl]).wait()
            r0 = rowbase(b)
            for i in range(BR):
                for m in range(8):
                    slc = pl.ds(m * 16, 16)
                    sidx[sl][i, slc] = srcv[sl][i, slc]
                    didx[sl][i, slc] = dstv[sl][i, slc]
            for j in range(BR):
                pltpu.async_copy(p_hbm.at[sidx[sl].at[j]],
                                 psb[sl].at[pl.ds(j * 128, 128)], semg[sl])
            for j in range(BR):
                pltpu.async_copy(q_hbm.at[didx[sl].at[j]],
                                 qdb[sl].at[pl.ds(j * 128, 128)], semg[sl])
            for kk in range(4):
                pltpu.async_copy(ea_hbm.at[kk, pl.ds(r0, BR)],
                                 eab[sl].at[kk], semg[sl])

        def wait_all(sl):
            pltpu.make_async_copy(p_hbm.at[pl.ds(0, BE)], psb[sl],
                                  semg[sl]).wait()
            pltpu.make_async_copy(q_hbm.at[pl.ds(0, BE)], qdb[sl],
                                  semg[sl]).wait()
            for kk in range(4):
                pltpu.make_async_copy(ea_hbm.at[kk, pl.ds(0, BR)],
                                      eab[sl].at[kk], semg[sl]).wait()

        def compute(b, sl):
            @pl.loop(0, BE // 16)
            def _(grp):
                res = jnp.zeros((16,), jnp.float32) + bm2s
                erow = grp // 8
                eoff = (grp % 8) * 16
                eav = [eab[sl][kk, erow, pl.ds(eoff, 16)] for kk in range(4)]
                for l in range(16):
                    e = grp * 16 + l
                    sks = [eav[kk][l] for kk in range(4)]
                    rvc = []
                    for t in range(4):
                        rv = bm1_vc[t]
                        for kk in range(4):
                            rv = rv + sks[kk] * wek[kk][t]
                        rvc.append(rv)
                    acc = None
                    for t in range(2):
                        sp = pl.ds(32 * t, 32)
                        sv = psb[sl][e, sp] + qdb[sl][e, sp]
                        ae, ao = plsc.unpack(
                            sv, format=plsc.PackFormat.INTERLEAVED)
                        ze = jnp.maximum(ae + rvc[2 * t], 0.0)
                        zo = jnp.maximum(ao + rvc[2 * t + 1], 0.0)
                        pa = ze * w_vc[2 * t] + zo * w_vc[2 * t + 1]
                        acc = pa if acc is None else acc + pa
                    res = jnp.where(lane == l, res + jnp.sum(acc), res)
                outv[sl][pl.ds(grp * 16, 16)] = res

            pltpu.sync_copy(outv[sl], out_hbm.at[pl.ds(rowbase(b) * 128, BE)])

        fire_idx(0, 0)
        fire(0, 0)
        fire_idx(1, 1)

        @pl.loop(0, HALF)
        def _(kk):
            b0 = kk * 2
            fire(b0 + 1, 1)
            not_last = kk < HALF - 1

            @pl.when(not_last)
            def _():
                fire_idx(b0 + 2, 0)

            wait_all(0)
            compute(b0, 0)

            @pl.when(not_last)
            def _():
                fire(b0 + 2, 0)
                fire_idx(b0 + 3, 1)

            wait_all(1)
            compute(b0 + 1, 1)

    return k(src2, dst2, pb, qb, eaT, wvec)


_PBLK = 512
_PGRID = NP // 2 // _PBLK


def _tc_prep(x8, dpair, W1):
    def body(x_ref, d_ref, w_ref, g_ref, dv_ref):
        w = w_ref[...]
        dL = lax.rsqrt(d_ref[:, 0:1] + 1.0)
        dR = lax.rsqrt(d_ref[:, 1:2] + 1.0)
        hL = jnp.dot(x_ref[:, 0:4], w, preferred_element_type=jnp.float32) * dL
        hR = jnp.dot(x_ref[:, 4:8], w, preferred_element_type=jnp.float32) * dR
        g_ref[...] = jnp.concatenate([hL, hR], axis=1)
        dv_ref[...] = jnp.concatenate([dL, dR], axis=1)

    return pl.pallas_call(
        body,
        grid=(_PGRID,),
        in_specs=[
            pl.BlockSpec((_PBLK, 8), lambda i: (i, 0)),
            pl.BlockSpec((_PBLK, 2), lambda i: (i, 0)),
            pl.BlockSpec((4, HID), lambda i: (0, 0)),
        ],
        out_specs=[
            pl.BlockSpec((_PBLK, 128), lambda i: (i, 0)),
            pl.BlockSpec((_PBLK, 2), lambda i: (i, 0)),
        ],
        out_shape=[
            jax.ShapeDtypeStruct((NP // 2, 128), jnp.float32),
            jax.ShapeDtypeStruct((NP // 2, 2), jnp.float32),
        ],
    )(x8, dpair, W1)


def _tc_combine(Sp, gw, dinv2, b_row, Wn):
    def body(s_ref, g_ref, d_ref, b_ref, w_ref, out_ref):
        w = w_ref[...]
        b = b_ref[...]
        dL = d_ref[:, 0:1]
        dR = d_ref[:, 1:2]
        hL = jnp.maximum(dL * (s_ref[:, 0:64] + g_ref[:, 0:64]) + b, 0.0)
        hR = jnp.maximum(dR * (s_ref[:, 64:128] + g_ref[:, 64:128]) + b, 0.0)
        oL = dL * jnp.dot(hL, w, preferred_element_type=jnp.float32)
        oR = dR * jnp.dot(hR, w, preferred_element_type=jnp.float32)
        out_ref[...] = jnp.concatenate([oL, oR], axis=1)

    return pl.pallas_call(
        body,
        grid=(_PGRID,),
        in_specs=[
            pl.BlockSpec((_PBLK, 128), lambda i: (i, 0)),
            pl.BlockSpec((_PBLK, 128), lambda i: (i, 0)),
            pl.BlockSpec((_PBLK, 2), lambda i: (i, 0)),
            pl.BlockSpec((1, HID), lambda i: (0, 0)),
            pl.BlockSpec((HID, HID), lambda i: (0, 0)),
        ],
        out_specs=pl.BlockSpec((_PBLK, 128), lambda i: (i, 0)),
        out_shape=jax.ShapeDtypeStruct((NP // 2, 128), jnp.float32),
    )(Sp, gw, dinv2, b_row, Wn)


def _tc_final_nodes(Sp, gw, dinv2, b_row, Wsrc, Wdst):
    def body(s_ref, g_ref, d_ref, b_ref, ws_ref, wd_ref,
             pl_ref, pr_ref, ql_ref, qr_ref):
        ws = ws_ref[...]
        wd = wd_ref[...]
        b = b_ref[...]
        dL = d_ref[:, 0:1]
        dR = d_ref[:, 1:2]
        hL = jnp.maximum(dL * (s_ref[:, 0:64] + g_ref[:, 0:64]) + b, 0.0)
        hR = jnp.maximum(dR * (s_ref[:, 64:128] + g_ref[:, 64:128]) + b, 0.0)
        pl_ref[...] = jnp.dot(hL, ws, preferred_element_type=jnp.float32
                              ).astype(jnp.bfloat16)
        pr_ref[...] = jnp.dot(hR, ws, preferred_element_type=jnp.float32
                              ).astype(jnp.bfloat16)
        ql_ref[...] = jnp.dot(hL, wd, preferred_element_type=jnp.float32
                              ).astype(jnp.bfloat16)
        qr_ref[...] = jnp.dot(hR, wd, preferred_element_type=jnp.float32
                              ).astype(jnp.bfloat16)

    return pl.pallas_call(
        body,
        grid=(_PGRID,),
        in_specs=[
            pl.BlockSpec((_PBLK, 128), lambda i: (i, 0)),
            pl.BlockSpec((_PBLK, 128), lambda i: (i, 0)),
            pl.BlockSpec((_PBLK, 2), lambda i: (i, 0)),
            pl.BlockSpec((1, HID), lambda i: (0, 0)),
            pl.BlockSpec((HID, HID), lambda i: (0, 0)),
            pl.BlockSpec((HID, HID), lambda i: (0, 0)),
        ],
        out_specs=[pl.BlockSpec((_PBLK, HID), lambda i: (i, 0))] * 4,
        out_shape=[jax.ShapeDtypeStruct((NP // 2, HID), jnp.bfloat16)] * 4,
    )(Sp, gw, dinv2, b_row, Wsrc, Wdst)


def kernel(x, edge_index, edge_attr, W1, b1, W2, b2, Wm1, bm1, Wm2, bm2):
    src = edge_index[0]
    dst = edge_index[1]
    src2 = jnp.full((EP,), N, jnp.int32).at[:E].set(src).reshape(ER, 128)
    dst2 = jnp.full((EP,), N, jnp.int32).at[:E].set(dst).reshape(ER, 128)
    eaT = (jnp.zeros((EP, 4), jnp.float32).at[:E].set(edge_attr)
           .T.reshape(4, ER, 128))
    x8 = jnp.zeros((NP, 4), jnp.float32).at[:N].set(x).reshape(NP // 2, 8)
    b1_row = b1.reshape(1, HID)
    b2_row = b2.reshape(1, HID)
    Wsrc = Wm1[0:HID]
    Wdst = Wm1[HID:2 * HID]
    perm = jnp.array(list(range(0, 32, 2)) + list(range(1, 32, 2))
                     + list(range(32, 64, 2)) + list(range(33, 64, 2)))
    Wep = Wm1[2 * HID:][:, perm]
    wvec = (jnp.zeros((392,), jnp.float32)
            .at[0:HID].set(Wm2[perm, 0])
            .at[HID:2 * HID].set(bm1[perm])
            .at[2 * HID:2 * HID + 256].set(Wep.reshape(-1))
            .at[384].set(bm2[0]))

    deg_parts = _sc_deg(dst2)
    dpair = (deg_parts[0, :, 0] + deg_parts[1, :, 0]).reshape(NP // 2, 2)
    gw1, dinv2 = _tc_prep(x8, dpair, W1)
    gflat1 = _sc_g_reshape(gw1)
    S1p = _sc_s_reshape(_sc_segsum(src2, dst2, gflat1))
    gw2 = _tc_combine(S1p, gw1, dinv2, b1_row, W2)
    gflat2 = _sc_g_reshape(gw2)
    S2p = _sc_s_reshape(_sc_segsum(src2, dst2, gflat2))
    pL, pR, qL, qR = _tc_final_nodes(S2p, gw2, dinv2, b2_row, Wsrc, Wdst)
    p = jnp.stack([pL, pR], axis=1).reshape(NP, HID)
    q = jnp.stack([qL, qR], axis=1).reshape(NP, HID)
    logits = _sc_mlp(src2, dst2, p, q, eaT, wvec)
    return logits[:E]
